# Initial kernel scaffold; baseline (speedup 1.0000x reference)
import jax
import jax.numpy as jnp
from jax.experimental import pallas as pl


def kernel(node, adj, batch, edge_attr, embedding, We, be, W_self1, W_neigh1, b1, W_self2, W_neigh2, b2, W_cls, b_cls):
    raise NotImplementedError("write your pallas kernel here")


# speedup vs baseline: 4.2955x; 4.2955x over previous
import functools

import jax
import jax.numpy as jnp
from jax import lax
from jax.experimental import pallas as pl
from jax.experimental.pallas import tpu as pltpu
from jax.experimental.pallas import tpu_sc as plsc

N_NODES = 100000
N_EDGES = 1600000
NUM_GRAPHS = 256
DIM = 50
DIMP = 64
CHUNK = 16
NCHUNK = DIMP // CHUNK

NC, NS = 2, 16
NW = NC * NS

NPAD = 102400
EBLK = 1024
EPT = 100352
EPAD = EPT * NS
BN = 512


def _embed_body(emb_hbm, node_hbm, xch_hbm, idx_v, rows_v, sem):
    core = lax.axis_index("c")
    sub = lax.axis_index("s")
    wid = sub * NC + core
    rpt = NPAD // NW

    def blk(b, _):
        base = pl.multiple_of(wid * rpt + b * 128, 128)
        pltpu.sync_copy(node_hbm.at[pl.ds(base, 128)], idx_v)
        pltpu.async_copy(emb_hbm.at[idx_v], rows_v, sem).wait()
        for c in range(NCHUNK):
            pltpu.sync_copy(rows_v.at[:, pl.ds(c * CHUNK, CHUNK)],
                            xch_hbm.at[pl.ds(c * NPAD + base, 128)])
        return 0

    lax.fori_loop(0, rpt // 128, blk, 0)


def _embed_call(embp, nodep):
    return pl.kernel(
        _embed_body,
        out_type=jax.ShapeDtypeStruct((NCHUNK * NPAD, CHUNK), jnp.float32),
        mesh=plsc.VectorSubcoreMesh(core_axis_name="c", subcore_axis_name="s"),
        compiler_params=pltpu.CompilerParams(use_tc_tiling_on_sc=False),
        scratch_types=[
            pltpu.VMEM((128,), jnp.int32),
            pltpu.VMEM((128, DIMP), jnp.float32),
            pltpu.SemaphoreType.DMA,
        ],
        name="sc_embed_gather",
    )(embp, nodep)


def _scatter_body(hch_hbm, src_hbm, dst_hbm, gate_hbm, agg_hbm,
                  agg_s, src_v, sidx_v, didx_v, gate_v, rows_v, zero_v,
                  sem, sem2):
    core = lax.axis_index("c")
    sub = lax.axis_index("s")
    rw = NPAD // NS

    def zinit(i, _):
        zero_v[i] = jnp.zeros((CHUNK,), jnp.float32)
        return 0
    lax.fori_loop(0, 128, zinit, 0)

    for cpass in range(NCHUNK // NC):
        chunk = core * (NCHUNK // NC) + cpass
        chunk_base = chunk * NPAD

        def zloop(i, _):
            pltpu.sync_copy(zero_v, agg_s.at[pl.ds(sub * rw + i * 128, 128)])
            return 0
        lax.fori_loop(0, rw // 128, zloop, 0)
        plsc.subcore_barrier()

        def eblock(b, _):
            ebase = pl.multiple_of(sub * EPT + b * EBLK, EBLK)
            erow = sub * (EPT // 128) + b * (EBLK // 128)
            pltpu.sync_copy(src_hbm.at[pl.ds(ebase, EBLK)], src_v)
            pltpu.sync_copy(dst_hbm.at[pl.ds(erow, EBLK // 128)], didx_v)
            pltpu.sync_copy(gate_hbm.at[pl.ds(ebase, EBLK)], gate_v)
            for g in range(EBLK // 16):
                r, o = divmod(g, 8)
                sidx_v[r, pl.ds(o * 16, 16)] = (
                    src_v[pl.ds(g * 16, 16)] + chunk_base)
            cps = [pltpu.async_copy(hch_hbm.at[sidx_v.at[r]],
                                    rows_v.at[pl.ds(r * 128, 128)], sem)
                   for r in range(EBLK // 128)]
            for cp in cps:
                cp.wait()

            def scale(k, _):
                gbase = pl.multiple_of(k * 16, 16)
                gv = gate_v[pl.ds(gbase, 16)]
                for r in range(16):
                    j = gbase + r
                    rows_v[j] = rows_v[j] * gv[r]
                return 0
            lax.fori_loop(0, EBLK // 16, scale, 0)

            cps = [pltpu.async_copy(rows_v.at[pl.ds(r * 128, 128)],
                                    agg_s.at[didx_v.at[r]], sem2, add=True)
                   for r in range(EBLK // 128)]
            for cp in cps:
                cp.wait()
            return 0
        lax.fori_loop(0, EPT // EBLK, eblock, 0)
        plsc.subcore_barrier()

        dst0 = pl.multiple_of(chunk_base + sub * rw, 128)
        pltpu.sync_copy(agg_s.at[pl.ds(sub * rw, rw)],
                        agg_hbm.at[pl.ds(dst0, rw)])
        plsc.subcore_barrier()


def _scatter_call(hch, srcp, dstp, gatep):
    return pl.kernel(
        _scatter_body,
        out_type=jax.ShapeDtypeStruct((NCHUNK * NPAD, CHUNK), jnp.float32),
        mesh=plsc.VectorSubcoreMesh(core_axis_name="c", subcore_axis_name="s"),
        compiler_params=pltpu.CompilerParams(use_tc_tiling_on_sc=False),
        scratch_types=[
            pltpu.VMEM_SHARED((NPAD, CHUNK), jnp.float32),
            pltpu.VMEM((EBLK,), jnp.int32),
            pltpu.VMEM((EBLK // 128, 128), jnp.int32),
            pltpu.VMEM((EBLK // 128, 128), jnp.int32),
            pltpu.VMEM((EBLK,), jnp.float32),
            pltpu.VMEM((EBLK, CHUNK), jnp.float32),
            pltpu.VMEM((128, CHUNK), jnp.float32),
            pltpu.SemaphoreType.DMA,
            pltpu.SemaphoreType.DMA,
        ],
        name="sc_edge_scatter",
    )(hch, srcp, dstp, gatep)


def _gate_body(ea_ref, wt_ref, be_ref, o_ref):
    t = ea_ref[...] * wt_ref[...]
    z = jnp.sum(t, axis=1, keepdims=True) + be_ref[...]
    o_ref[...] = 1.0 / (1.0 + jnp.exp(-z))


def _gate_call(edge_attr, WeT, be2):
    EB = 8000
    return pl.pallas_call(
        _gate_body,
        grid=(N_EDGES // EB,),
        in_specs=[
            pl.BlockSpec((EB, 4), lambda i: (i, 0)),
            pl.BlockSpec((1, 4), lambda i: (0, 0)),
            pl.BlockSpec((1, 1), lambda i: (0, 0)),
        ],
        out_specs=pl.BlockSpec((EB, 1), lambda i: (i, 0)),
        out_shape=jax.ShapeDtypeStruct((N_EDGES, 1), jnp.float32),
    )(edge_attr, WeT, be2)


def _layer_body(h_ref, a_ref, ws_ref, wn_ref, b_ref, o_ref):
    h = jnp.concatenate([h_ref[c] for c in range(NCHUNK)], axis=-1)
    a = jnp.concatenate([a_ref[c] for c in range(NCHUNK)], axis=-1)
    denom = a[:, DIM:DIM + 1] + 1e-6
    an = a / denom
    z = (jnp.dot(h, ws_ref[...], preferred_element_type=jnp.float32)
         + jnp.dot(an, wn_ref[...], preferred_element_type=jnp.float32)
         + b_ref[...])
    z = jnp.maximum(z, 0.0)
    lanes = lax.broadcasted_iota(jnp.int32, (BN, DIMP), 1)
    z = jnp.where(lanes == DIM, 1.0, z)
    for c in range(NCHUNK):
        o_ref[c] = z[:, c * CHUNK:(c + 1) * CHUNK]


def _layer_call(hch4, aggch4, Wsp, Wnp, bp):
    return pl.pallas_call(
        _layer_body,
        grid=(NPAD // BN,),
        in_specs=[
            pl.BlockSpec((NCHUNK, BN, CHUNK), lambda i: (0, i, 0)),
            pl.BlockSpec((NCHUNK, BN, CHUNK), lambda i: (0, i, 0)),
            pl.BlockSpec((DIMP, DIMP), lambda i: (0, 0)),
            pl.BlockSpec((DIMP, DIMP), lambda i: (0, 0)),
            pl.BlockSpec((1, DIMP), lambda i: (0, 0)),
        ],
        out_specs=pl.BlockSpec((NCHUNK, BN, CHUNK), lambda i: (0, i, 0)),
        out_shape=jax.ShapeDtypeStruct((NCHUNK, NPAD, CHUNK), jnp.float32),
    )(hch4, aggch4, Wsp, Wnp, bp)


def _pool_body(h_ref, b_ref, o_ref):
    i = pl.program_id(0)
    h = jnp.concatenate([h_ref[c] for c in range(NCHUNK)], axis=-1)
    bid = b_ref[0]
    ohT = (lax.broadcasted_iota(jnp.int32, (NUM_GRAPHS, BN), 0)
           == bid).astype(jnp.float32)
    part = jnp.dot(ohT, h, preferred_element_type=jnp.float32)

    @pl.when(i == 0)
    def _():
        o_ref[...] = part

    @pl.when(i > 0)
    def _():
        o_ref[...] += part


def _pool_call(hch4, batch3):
    return pl.pallas_call(
        _pool_body,
        grid=(NPAD // BN,),
        in_specs=[
            pl.BlockSpec((NCHUNK, BN, CHUNK), lambda i: (0, i, 0)),
            pl.BlockSpec((1, 1, BN), lambda i: (i, 0, 0)),
        ],
        out_specs=pl.BlockSpec((NUM_GRAPHS, DIMP), lambda i: (0, 0)),
        out_shape=jax.ShapeDtypeStruct((NUM_GRAPHS, DIMP), jnp.float32),
    )(hch4, batch3)


def _logits_body(p_ref, wc_ref, bc_ref, o_ref):
    ps = p_ref[...]
    counts = ps[:, DIM:DIM + 1] + 1e-6
    pm = ps / counts
    o_ref[...] = (jnp.dot(pm, wc_ref[...], preferred_element_type=jnp.float32)
                  + bc_ref[...])


def _logits_call(pooled, Wcp, bcp):
    ncls = Wcp.shape[1]
    return pl.pallas_call(
        _logits_body,
        grid=(1,),
        in_specs=[
            pl.BlockSpec((NUM_GRAPHS, DIMP), lambda i: (0, 0)),
            pl.BlockSpec((DIMP, ncls), lambda i: (0, 0)),
            pl.BlockSpec((1, ncls), lambda i: (0, 0)),
        ],
        out_specs=pl.BlockSpec((NUM_GRAPHS, ncls), lambda i: (0, 0)),
        out_shape=jax.ShapeDtypeStruct((NUM_GRAPHS, ncls), jnp.float32),
    )(pooled, Wcp, bcp)


def _pad64(w):
    return jnp.pad(w, ((0, DIMP - DIM), (0, DIMP - DIM)))


def kernel(node, adj, batch, edge_attr, embedding, We, be,
           W_self1, W_neigh1, b1, W_self2, W_neigh2, b2, W_cls, b_cls):
    V = embedding.shape[0]
    src = adj[0]
    dst = adj[1]

    i32 = jnp.int32
    nodep = jnp.concatenate([node, jnp.zeros((NPAD - N_NODES,), i32)])
    srcp = jnp.concatenate([src, jnp.zeros((EPAD - N_EDGES,), i32)])
    dstp = jnp.concatenate(
        [dst, jnp.zeros((EPAD - N_EDGES,), i32)]).reshape(EPAD // 128, 128)
    batchp = jnp.concatenate(
        [batch, jnp.full((NPAD - N_NODES,), NUM_GRAPHS, i32)])
    batch3 = batchp.reshape(NPAD // BN, 1, BN)

    embp = jnp.concatenate(
        [embedding, jnp.ones((V, 1), jnp.float32),
         jnp.zeros((V, DIMP - DIM - 1), jnp.float32)], axis=1)

    Wsp1, Wnp1 = _pad64(W_self1), _pad64(W_neigh1)
    Wsp2, Wnp2 = _pad64(W_self2), _pad64(W_neigh2)
    bp1 = jnp.pad(b1, (0, DIMP - DIM)).reshape(1, DIMP)
    bp2 = jnp.pad(b2, (0, DIMP - DIM)).reshape(1, DIMP)
    Wcp = jnp.pad(W_cls, ((0, DIMP - DIM), (0, 0)))
    bcp = b_cls.reshape(1, -1)
    WeT = We.reshape(1, -1)
    be2 = be.reshape(1, 1)

    gate2 = _gate_call(edge_attr, WeT, be2)
    gatep = jnp.concatenate(
        [gate2.reshape(N_EDGES), jnp.zeros((EPAD - N_EDGES,), jnp.float32)])

    hch = _embed_call(embp, nodep)
    for Wsp, Wnp, bp in ((Wsp1, Wnp1, bp1), (Wsp2, Wnp2, bp2)):
        aggch = _scatter_call(hch, srcp, dstp, gatep)
        hch = _layer_call(hch.reshape(NCHUNK, NPAD, CHUNK),
                          aggch.reshape(NCHUNK, NPAD, CHUNK),
                          Wsp, Wnp, bp).reshape(NCHUNK * NPAD, CHUNK)

    pooled = _pool_call(hch.reshape(NCHUNK, NPAD, CHUNK), batch3)
    return _logits_call(pooled, Wcp, bcp)


# speedup vs baseline: 4.9680x; 1.1566x over previous
import jax
import jax.numpy as jnp
from jax import lax
from jax.experimental import pallas as pl
from jax.experimental.pallas import tpu as pltpu
from jax.experimental.pallas import tpu_sc as plsc

N_NODES = 100000
N_EDGES = 1600000
NUM_GRAPHS = 256
DIM = 50
DIMP = 64
CHUNK = 16
NCHUNK = DIMP // CHUNK
SLOTS = 128 // CHUNK

NC, NS = 2, 16
NW = NC * NS

NPAD = 102400
EBLK = 1024
EPT = 100352
EPAD = EPT * NS
ETILE = N_EDGES // 128
GROWS = EPAD // NW
BN = 512


def _prep_body(emb_hbm, node_hbm, ea_hbm, web_hbm, x3_hbm, gate_hbm,
               idx_v, rows_v, ea_v, g_v, w_v, sem):
    core = lax.axis_index("c")
    sub = lax.axis_index("s")
    wid = sub * NC + core
    rpt = NPAD // NW

    def blk(b, _):
        base = pl.multiple_of(wid * rpt + b * 128, 128)
        pltpu.sync_copy(node_hbm.at[pl.ds(base, 128)], idx_v)
        pltpu.async_copy(emb_hbm.at[idx_v], rows_v, sem).wait()
        for c in range(NCHUNK):
            pltpu.sync_copy(rows_v.at[:, pl.ds(c * CHUNK, CHUNK)],
                            x3_hbm.at[pl.ds(base, 128), c])
        return 0
    lax.fori_loop(0, rpt // 128, blk, 0)

    pltpu.sync_copy(web_hbm, w_v)
    wv = w_v[...]
    ert = GROWS // 128
    row0 = wid * ert
    nrow = jnp.maximum(jnp.minimum(ETILE - row0, ert), 0)

    def grow(i, _):
        r = row0 + i
        pltpu.sync_copy(ea_hbm.at[r], ea_v)
        for g in range(8):
            z = (ea_v[0, pl.ds(g * 16, 16)] * wv[0]
                 + ea_v[1, pl.ds(g * 16, 16)] * wv[1]
                 + ea_v[2, pl.ds(g * 16, 16)] * wv[2]
                 + ea_v[3, pl.ds(g * 16, 16)] * wv[3]
                 + wv[4])
            g_v[pl.ds(g * 16, 16)] = 1.0 / (1.0 + jnp.exp(-z))
        gbase = pl.multiple_of((row0 + i) * 128, 128)
        pltpu.sync_copy(g_v, gate_hbm.at[pl.ds(gbase, 128)])
        return 0
    lax.fori_loop(0, nrow, grow, 0)

    @pl.when(wid == NW - 1)
    def _():
        def ztail(g, _):
            g_v[pl.ds(g * 16, 16)] = jnp.zeros((16,), jnp.float32)
            return 0
        lax.fori_loop(0, 8, ztail, 0)

        def zrow(i, _):
            gbase = pl.multiple_of(N_EDGES + i * 128, 128)
            pltpu.sync_copy(g_v, gate_hbm.at[pl.ds(gbase, 128)])
            return 0
        lax.fori_loop(0, (EPAD - N_EDGES) // 128, zrow, 0)


def _prep_call(embp, nodep, ea3, web):
    return pl.kernel(
        _prep_body,
        out_type=(
            jax.ShapeDtypeStruct((NPAD, SLOTS, CHUNK), jnp.float32),
            jax.ShapeDtypeStruct((EPAD,), jnp.float32),
        ),
        mesh=plsc.VectorSubcoreMesh(core_axis_name="c", subcore_axis_name="s"),
        compiler_params=pltpu.CompilerParams(use_tc_tiling_on_sc=False),
        scratch_types=[
            pltpu.VMEM((128,), jnp.int32),
            pltpu.VMEM((128, DIMP), jnp.float32),
            pltpu.VMEM((4, 128), jnp.float32),
            pltpu.VMEM((128,), jnp.float32),
            pltpu.VMEM((16,), jnp.float32),
            pltpu.SemaphoreType.DMA,
        ],
        name="sc_prep",
    )(embp, nodep, ea3, web)


def _scatter_body(h_hbm, src_hbm, dst_hbm, gate_hbm, agg_hbm,
                  agg_s, src_v, sidx_v, didx_v, gate_v, rows_v, zero_v,
                  sem, sem2):
    core = lax.axis_index("c")
    sub = lax.axis_index("s")
    rw = NPAD // NS

    def zinit(i, _):
        zero_v[i] = jnp.zeros((CHUNK,), jnp.float32)
        return 0
    lax.fori_loop(0, 128, zinit, 0)

    for cpass in range(NCHUNK // NC):
        chunk = core * (NCHUNK // NC) + cpass

        def zloop(i, _):
            pltpu.sync_copy(zero_v, agg_s.at[pl.ds(sub * rw + i * 128, 128)])
            return 0
        lax.fori_loop(0, rw // 128, zloop, 0)
        plsc.subcore_barrier()

        def eblock(b, _):
            ebase = pl.multiple_of(sub * EPT + b * EBLK, EBLK)
            erow = sub * (EPT // 128) + b * (EBLK // 128)
            pltpu.sync_copy(src_hbm.at[pl.ds(ebase, EBLK)], src_v)
            pltpu.sync_copy(dst_hbm.at[pl.ds(erow, EBLK // 128)], didx_v)
            pltpu.sync_copy(gate_hbm.at[pl.ds(ebase, EBLK)], gate_v)
            for g in range(EBLK // 16):
                r, o = divmod(g, 8)
                sidx_v[r, pl.ds(o * 16, 16)] = (
                    src_v[pl.ds(g * 16, 16)] * SLOTS + chunk)
            cps = [pltpu.async_copy(h_hbm.at[sidx_v.at[r]],
                                    rows_v.at[pl.ds(r * 128, 128)], sem)
                   for r in range(EBLK // 128)]
            for cp in cps:
                cp.wait()

            def scale(k, _):
                gbase = pl.multiple_of(k * 16, 16)
                gv = gate_v[pl.ds(gbase, 16)]
                for r in range(16):
                    j = gbase + r
                    rows_v[j] = rows_v[j] * gv[r]
                return 0
            lax.fori_loop(0, EBLK // 16, scale, 0)

            cps = [pltpu.async_copy(rows_v.at[pl.ds(r * 128, 128)],
                                    agg_s.at[didx_v.at[r]], sem2, add=True)
                   for r in range(EBLK // 128)]
            for cp in cps:
                cp.wait()
            return 0
        lax.fori_loop(0, EPT // EBLK, eblock, 0)
        plsc.subcore_barrier()

        r0 = pl.multiple_of(sub * rw, 128)
        pltpu.sync_copy(agg_s.at[pl.ds(sub * rw, rw)],
                        agg_hbm.at[pl.ds(r0, rw), chunk])
        plsc.subcore_barrier()


def _scatter_call(h128, srcp, dstp, gatep):
    hrows = h128.reshape(SLOTS * NPAD, CHUNK)
    return pl.kernel(
        _scatter_body,
        out_type=jax.ShapeDtypeStruct((NPAD, SLOTS, CHUNK), jnp.float32),
        mesh=plsc.VectorSubcoreMesh(core_axis_name="c", subcore_axis_name="s"),
        compiler_params=pltpu.CompilerParams(use_tc_tiling_on_sc=False),
        scratch_types=[
            pltpu.VMEM_SHARED((NPAD, CHUNK), jnp.float32),
            pltpu.VMEM((EBLK,), jnp.int32),
            pltpu.VMEM((EBLK // 128, 128), jnp.int32),
            pltpu.VMEM((EBLK // 128, 128), jnp.int32),
            pltpu.VMEM((EBLK,), jnp.float32),
            pltpu.VMEM((EBLK, CHUNK), jnp.float32),
            pltpu.VMEM((128, CHUNK), jnp.float32),
            pltpu.SemaphoreType.DMA,
            pltpu.SemaphoreType.DMA,
        ],
        name="sc_edge_scatter",
    )(hrows, srcp, dstp, gatep)


def _layer_body(h_ref, a_ref, ws_ref, wn_ref, b_ref, o_ref):
    h = h_ref[...][:, :DIMP]
    a = a_ref[...][:, :DIMP]
    denom = a[:, DIM:DIM + 1] + 1e-6
    an = a / denom
    z = (jnp.dot(h, ws_ref[...], preferred_element_type=jnp.float32)
         + jnp.dot(an, wn_ref[...], preferred_element_type=jnp.float32)
         + b_ref[...])
    z = jnp.maximum(z, 0.0)
    lanes = lax.broadcasted_iota(jnp.int32, (BN, DIMP), 1)
    z = jnp.where(lanes == DIM, 1.0, z)
    o_ref[...] = jnp.concatenate(
        [z, jnp.zeros((BN, 128 - DIMP), jnp.float32)], axis=1)


def _layer_call(h128, agg128, Wsp, Wnp, bp):
    return pl.pallas_call(
        _layer_body,
        grid=(NPAD // BN,),
        in_specs=[
            pl.BlockSpec((BN, 128), lambda i: (i, 0)),
            pl.BlockSpec((BN, 128), lambda i: (i, 0)),
            pl.BlockSpec((DIMP, DIMP), lambda i: (0, 0)),
            pl.BlockSpec((DIMP, DIMP), lambda i: (0, 0)),
            pl.BlockSpec((1, DIMP), lambda i: (0, 0)),
        ],
        out_specs=pl.BlockSpec((BN, 128), lambda i: (i, 0)),
        out_shape=jax.ShapeDtypeStruct((NPAD, 128), jnp.float32),
    )(h128, agg128, Wsp, Wnp, bp)


def _pool_body(h_ref, b_ref, o_ref):
    i = pl.program_id(0)
    h = h_ref[...][:, :DIMP]
    bid = b_ref[0]
    ohT = (lax.broadcasted_iota(jnp.int32, (NUM_GRAPHS, BN), 0)
           == bid).astype(jnp.float32)
    part = jnp.dot(ohT, h, preferred_element_type=jnp.float32)

    @pl.when(i == 0)
    def _():
        o_ref[...] = part

    @pl.when(i > 0)
    def _():
        o_ref[...] += part


def _pool_call(h128, batch3):
    return pl.pallas_call(
        _pool_body,
        grid=(NPAD // BN,),
        in_specs=[
            pl.BlockSpec((BN, 128), lambda i: (i, 0)),
            pl.BlockSpec((1, 1, BN), lambda i: (i, 0, 0)),
        ],
        out_specs=pl.BlockSpec((NUM_GRAPHS, DIMP), lambda i: (0, 0)),
        out_shape=jax.ShapeDtypeStruct((NUM_GRAPHS, DIMP), jnp.float32),
    )(h128, batch3)


def _logits_body(p_ref, wc_ref, bc_ref, o_ref):
    ps = p_ref[...]
    counts = ps[:, DIM:DIM + 1] + 1e-6
    pm = ps / counts
    o_ref[...] = (jnp.dot(pm, wc_ref[...], preferred_element_type=jnp.float32)
                  + bc_ref[...])


def _logits_call(pooled, Wcp, bcp):
    ncls = Wcp.shape[1]
    return pl.pallas_call(
        _logits_body,
        grid=(1,),
        in_specs=[
            pl.BlockSpec((NUM_GRAPHS, DIMP), lambda i: (0, 0)),
            pl.BlockSpec((DIMP, ncls), lambda i: (0, 0)),
            pl.BlockSpec((1, ncls), lambda i: (0, 0)),
        ],
        out_specs=pl.BlockSpec((NUM_GRAPHS, ncls), lambda i: (0, 0)),
        out_shape=jax.ShapeDtypeStruct((NUM_GRAPHS, ncls), jnp.float32),
    )(pooled, Wcp, bcp)


def _pad64(w):
    return jnp.pad(w, ((0, DIMP - DIM), (0, DIMP - DIM)))


def kernel(node, adj, batch, edge_attr, embedding, We, be,
           W_self1, W_neigh1, b1, W_self2, W_neigh2, b2, W_cls, b_cls):
    V = embedding.shape[0]
    src = adj[0]
    dst = adj[1]

    i32 = jnp.int32
    nodep = jnp.concatenate([node, jnp.zeros((NPAD - N_NODES,), i32)])
    srcp = jnp.concatenate([src, jnp.zeros((EPAD - N_EDGES,), i32)])
    dstp = jnp.concatenate(
        [dst, jnp.zeros((EPAD - N_EDGES,), i32)]).reshape(EPAD // 128, 128)
    batchp = jnp.concatenate(
        [batch, jnp.full((NPAD - N_NODES,), NUM_GRAPHS, i32)])
    batch3 = batchp.reshape(NPAD // BN, 1, BN)

    embp = jnp.concatenate(
        [embedding, jnp.ones((V, 1), jnp.float32),
         jnp.zeros((V, DIMP - DIM - 1), jnp.float32)], axis=1)
    ea3 = edge_attr.reshape(ETILE, 128, 4).transpose(0, 2, 1)
    web = jnp.concatenate(
        [We.reshape(-1), be, jnp.zeros((16 - 5,), jnp.float32)])

    Wsp1, Wnp1 = _pad64(W_self1), _pad64(W_neigh1)
    Wsp2, Wnp2 = _pad64(W_self2), _pad64(W_neigh2)
    bp1 = jnp.pad(b1, (0, DIMP - DIM)).reshape(1, DIMP)
    bp2 = jnp.pad(b2, (0, DIMP - DIM)).reshape(1, DIMP)
    Wcp = jnp.pad(W_cls, ((0, DIMP - DIM), (0, 0)))
    bcp = b_cls.reshape(1, -1)

    x3, gatep = _prep_call(embp, nodep, ea3, web)
    h128 = x3.reshape(NPAD, 128)
    for Wsp, Wnp, bp in ((Wsp1, Wnp1, bp1), (Wsp2, Wnp2, bp2)):
        agg3 = _scatter_call(h128, srcp, dstp, gatep)
        h128 = _layer_call(h128, agg3.reshape(NPAD, 128), Wsp, Wnp, bp)

    pooled = _pool_call(h128, batch3)
    return _logits_call(pooled, Wcp, bcp)


# speedup vs baseline: 6.3722x; 1.2826x over previous
import jax
import jax.numpy as jnp
from jax import lax
from jax.experimental import pallas as pl
from jax.experimental.pallas import tpu as pltpu
from jax.experimental.pallas import tpu_sc as plsc

N_NODES = 100000
N_EDGES = 1600000
NUM_GRAPHS = 256
DIM = 50
DIMP = 64
CHUNK = 16
NCHUNK = DIMP // CHUNK
SLOTS = 128 // CHUNK

NC, NS = 2, 16
NW = NC * NS

NPAD = 102400
EBLK = 1024
EPT = 100352
EPAD = EPT * NS
ETILE = N_EDGES // 128
GROWS = EPAD // NW
BN = 512


def _prep_body(emb_hbm, node_hbm, ea_hbm, web_hbm, x3_hbm, gate_hbm,
               idx_v, rows_v, ea_v, g_v, w_v, sem):
    core = lax.axis_index("c")
    sub = lax.axis_index("s")
    wid = sub * NC + core
    rpt = NPAD // NW

    def blk(b, _):
        base = pl.multiple_of(wid * rpt + b * 128, 128)
        pltpu.sync_copy(node_hbm.at[pl.ds(base, 128)], idx_v)
        pltpu.async_copy(emb_hbm.at[idx_v], rows_v, sem).wait()
        pltpu.sync_copy(rows_v, x3_hbm.at[pl.ds(base, 128), pl.ds(0, DIMP)])
        return 0
    lax.fori_loop(0, rpt // 128, blk, 0)

    pltpu.sync_copy(web_hbm, w_v)
    wv = w_v[...]
    ert = GROWS // 128
    row0 = wid * ert
    nrow = jnp.maximum(jnp.minimum(ETILE - row0, ert), 0)

    def grow(i, _):
        r = row0 + i
        pltpu.sync_copy(ea_hbm.at[r], ea_v)
        for g in range(8):
            z = (ea_v[0, pl.ds(g * 16, 16)] * wv[0]
                 + ea_v[1, pl.ds(g * 16, 16)] * wv[1]
                 + ea_v[2, pl.ds(g * 16, 16)] * wv[2]
                 + ea_v[3, pl.ds(g * 16, 16)] * wv[3]
                 + wv[4])
            g_v[pl.ds(g * 16, 16)] = 1.0 / (1.0 + jnp.exp(-z))
        gbase = pl.multiple_of((row0 + i) * 128, 128)
        pltpu.sync_copy(g_v, gate_hbm.at[pl.ds(gbase, 128)])
        return 0
    lax.fori_loop(0, nrow, grow, 0)

    @pl.when(wid == NW - 1)
    def _():
        def ztail(g, _):
            g_v[pl.ds(g * 16, 16)] = jnp.zeros((16,), jnp.float32)
            return 0
        lax.fori_loop(0, 8, ztail, 0)

        def zrow(i, _):
            gbase = pl.multiple_of(N_EDGES + i * 128, 128)
            pltpu.sync_copy(g_v, gate_hbm.at[pl.ds(gbase, 128)])
            return 0
        lax.fori_loop(0, (EPAD - N_EDGES) // 128, zrow, 0)


def _prep_call(embp, nodep, ea3, web):
    return pl.kernel(
        _prep_body,
        out_type=(
            jax.ShapeDtypeStruct((NPAD, 128), jnp.float32),
            jax.ShapeDtypeStruct((EPAD,), jnp.float32),
        ),
        mesh=plsc.VectorSubcoreMesh(core_axis_name="c", subcore_axis_name="s"),
        compiler_params=pltpu.CompilerParams(use_tc_tiling_on_sc=False),
        scratch_types=[
            pltpu.VMEM((128,), jnp.int32),
            pltpu.VMEM((128, DIMP), jnp.float32),
            pltpu.VMEM((4, 128), jnp.float32),
            pltpu.VMEM((128,), jnp.float32),
            pltpu.VMEM((16,), jnp.float32),
            pltpu.SemaphoreType.DMA,
        ],
        name="sc_prep",
    )(embp, nodep, ea3, web)


def _scatter_body(h_hbm, src_hbm, dst_hbm, gate_hbm, agg_hbm,
                  agg_s, src_v, sidx_v, didx_v, gate_v, rows_v, zero_v,
                  sem, sem2):
    core = lax.axis_index("c")
    sub = lax.axis_index("s")
    rw = NPAD // NS

    def zinit(i, _):
        zero_v[i] = jnp.zeros((CHUNK,), jnp.float32)
        return 0
    lax.fori_loop(0, 128, zinit, 0)

    for cpass in range(NCHUNK // NC):
        chunk = core * (NCHUNK // NC) + cpass

        def zloop(i, _):
            pltpu.sync_copy(zero_v, agg_s.at[pl.ds(sub * rw + i * 128, 128)])
            return 0
        lax.fori_loop(0, rw // 128, zloop, 0)
        plsc.subcore_barrier()

        def eblock(b, _):
            ebase = pl.multiple_of(sub * EPT + b * EBLK, EBLK)
            erow = sub * (EPT // 128) + b * (EBLK // 128)
            pltpu.sync_copy(src_hbm.at[pl.ds(ebase, EBLK)], src_v)
            pltpu.sync_copy(dst_hbm.at[pl.ds(erow, EBLK // 128)], didx_v)
            pltpu.sync_copy(gate_hbm.at[pl.ds(ebase, EBLK)], gate_v)
            for g in range(EBLK // 16):
                r, o = divmod(g, 8)
                sidx_v[r, pl.ds(o * 16, 16)] = (
                    src_v[pl.ds(g * 16, 16)] * SLOTS + chunk)
            cps = [pltpu.async_copy(h_hbm.at[sidx_v.at[r]],
                                    rows_v.at[pl.ds(r * 128, 128)], sem)
                   for r in range(EBLK // 128)]
            for cp in cps:
                cp.wait()

            def scale(k, _):
                gbase = pl.multiple_of(k * 16, 16)
                gv = gate_v[pl.ds(gbase, 16)]
                for r in range(16):
                    j = gbase + r
                    rows_v[j] = rows_v[j] * gv[r]
                return 0
            lax.fori_loop(0, EBLK // 16, scale, 0)

            cps = [pltpu.async_copy(rows_v.at[pl.ds(r * 128, 128)],
                                    agg_s.at[didx_v.at[r]], sem2, add=True)
                   for r in range(EBLK // 128)]
            for cp in cps:
                cp.wait()
            return 0
        lax.fori_loop(0, EPT // EBLK, eblock, 0)
        plsc.subcore_barrier()

        r0 = pl.multiple_of(sub * rw, 128)
        c0 = pl.multiple_of(chunk * CHUNK, CHUNK)
        pltpu.sync_copy(agg_s.at[pl.ds(sub * rw, rw)],
                        agg_hbm.at[pl.ds(r0, rw), pl.ds(c0, CHUNK)])
        plsc.subcore_barrier()


def _scatter_call(h128, srcp, dstp, gatep):
    hrows = h128.reshape(SLOTS * NPAD, CHUNK)
    return pl.kernel(
        _scatter_body,
        out_type=jax.ShapeDtypeStruct((NPAD, 128), jnp.float32),
        mesh=plsc.VectorSubcoreMesh(core_axis_name="c", subcore_axis_name="s"),
        compiler_params=pltpu.CompilerParams(use_tc_tiling_on_sc=False),
        scratch_types=[
            pltpu.VMEM_SHARED((NPAD, CHUNK), jnp.float32),
            pltpu.VMEM((EBLK,), jnp.int32),
            pltpu.VMEM((EBLK // 128, 128), jnp.int32),
            pltpu.VMEM((EBLK // 128, 128), jnp.int32),
            pltpu.VMEM((EBLK,), jnp.float32),
            pltpu.VMEM((EBLK, CHUNK), jnp.float32),
            pltpu.VMEM((128, CHUNK), jnp.float32),
            pltpu.SemaphoreType.DMA,
            pltpu.SemaphoreType.DMA,
        ],
        name="sc_edge_scatter",
    )(hrows, srcp, dstp, gatep)


def _layer_body(h_ref, a_ref, ws_ref, wn_ref, b_ref, o_ref):
    h = h_ref[...][:, :DIMP]
    a = a_ref[...][:, :DIMP]
    denom = a[:, DIM:DIM + 1] + 1e-6
    an = a / denom
    z = (jnp.dot(h, ws_ref[...], preferred_element_type=jnp.float32)
         + jnp.dot(an, wn_ref[...], preferred_element_type=jnp.float32)
         + b_ref[...])
    z = jnp.maximum(z, 0.0)
    lanes = lax.broadcasted_iota(jnp.int32, (BN, DIMP), 1)
    z = jnp.where(lanes == DIM, 1.0, z)
    o_ref[...] = jnp.concatenate(
        [z, jnp.zeros((BN, 128 - DIMP), jnp.float32)], axis=1)


def _layer_call(h128, agg128, Wsp, Wnp, bp):
    return pl.pallas_call(
        _layer_body,
        grid=(NPAD // BN,),
        in_specs=[
            pl.BlockSpec((BN, 128), lambda i: (i, 0)),
            pl.BlockSpec((BN, 128), lambda i: (i, 0)),
            pl.BlockSpec((DIMP, DIMP), lambda i: (0, 0)),
            pl.BlockSpec((DIMP, DIMP), lambda i: (0, 0)),
            pl.BlockSpec((1, DIMP), lambda i: (0, 0)),
        ],
        out_specs=pl.BlockSpec((BN, 128), lambda i: (i, 0)),
        out_shape=jax.ShapeDtypeStruct((NPAD, 128), jnp.float32),
    )(h128, agg128, Wsp, Wnp, bp)


def _pool_body(h_ref, b_ref, o_ref):
    i = pl.program_id(0)
    h = h_ref[...][:, :DIMP]
    bid = b_ref[0]
    ohT = (lax.broadcasted_iota(jnp.int32, (NUM_GRAPHS, BN), 0)
           == bid).astype(jnp.float32)
    part = jnp.dot(ohT, h, preferred_element_type=jnp.float32)

    @pl.when(i == 0)
    def _():
        o_ref[...] = part

    @pl.when(i > 0)
    def _():
        o_ref[...] += part


def _pool_call(h128, batch3):
    return pl.pallas_call(
        _pool_body,
        grid=(NPAD // BN,),
        in_specs=[
            pl.BlockSpec((BN, 128), lambda i: (i, 0)),
            pl.BlockSpec((1, 1, BN), lambda i: (i, 0, 0)),
        ],
        out_specs=pl.BlockSpec((NUM_GRAPHS, DIMP), lambda i: (0, 0)),
        out_shape=jax.ShapeDtypeStruct((NUM_GRAPHS, DIMP), jnp.float32),
    )(h128, batch3)


def _logits_body(p_ref, wc_ref, bc_ref, o_ref):
    ps = p_ref[...]
    counts = ps[:, DIM:DIM + 1] + 1e-6
    pm = ps / counts
    o_ref[...] = (jnp.dot(pm, wc_ref[...], preferred_element_type=jnp.float32)
                  + bc_ref[...])


def _logits_call(pooled, Wcp, bcp):
    ncls = Wcp.shape[1]
    return pl.pallas_call(
        _logits_body,
        grid=(1,),
        in_specs=[
            pl.BlockSpec((NUM_GRAPHS, DIMP), lambda i: (0, 0)),
            pl.BlockSpec((DIMP, ncls), lambda i: (0, 0)),
            pl.BlockSpec((1, ncls), lambda i: (0, 0)),
        ],
        out_specs=pl.BlockSpec((NUM_GRAPHS, ncls), lambda i: (0, 0)),
        out_shape=jax.ShapeDtypeStruct((NUM_GRAPHS, ncls), jnp.float32),
    )(pooled, Wcp, bcp)


def _pad64(w):
    return jnp.pad(w, ((0, DIMP - DIM), (0, DIMP - DIM)))


def kernel(node, adj, batch, edge_attr, embedding, We, be,
           W_self1, W_neigh1, b1, W_self2, W_neigh2, b2, W_cls, b_cls):
    V = embedding.shape[0]
    src = adj[0]
    dst = adj[1]

    i32 = jnp.int32
    nodep = jnp.concatenate([node, jnp.zeros((NPAD - N_NODES,), i32)])
    srcp = jnp.concatenate([src, jnp.zeros((EPAD - N_EDGES,), i32)])
    dstp = jnp.concatenate(
        [dst, jnp.zeros((EPAD - N_EDGES,), i32)]).reshape(EPAD // 128, 128)
    batchp = jnp.concatenate(
        [batch, jnp.full((NPAD - N_NODES,), NUM_GRAPHS, i32)])
    batch3 = batchp.reshape(NPAD // BN, 1, BN)

    embp = jnp.concatenate(
        [embedding, jnp.ones((V, 1), jnp.float32),
         jnp.zeros((V, DIMP - DIM - 1), jnp.float32)], axis=1)
    ea3 = edge_attr.reshape(ETILE, 128, 4).transpose(0, 2, 1)
    web = jnp.concatenate(
        [We.reshape(-1), be, jnp.zeros((16 - 5,), jnp.float32)])

    Wsp1, Wnp1 = _pad64(W_self1), _pad64(W_neigh1)
    Wsp2, Wnp2 = _pad64(W_self2), _pad64(W_neigh2)
    bp1 = jnp.pad(b1, (0, DIMP - DIM)).reshape(1, DIMP)
    bp2 = jnp.pad(b2, (0, DIMP - DIM)).reshape(1, DIMP)
    Wcp = jnp.pad(W_cls, ((0, DIMP - DIM), (0, 0)))
    bcp = b_cls.reshape(1, -1)

    h128, gatep = _prep_call(embp, nodep, ea3, web)
    for Wsp, Wnp, bp in ((Wsp1, Wnp1, bp1), (Wsp2, Wnp2, bp2)):
        agg128 = _scatter_call(h128, srcp, dstp, gatep)
        h128 = _layer_call(h128, agg128, Wsp, Wnp, bp)

    pooled = _pool_call(h128, batch3)
    return _logits_call(pooled, Wcp, bcp)


# speedup vs baseline: 7.3097x; 1.1471x over previous
import jax
import jax.numpy as jnp
from jax import lax
from jax.experimental import pallas as pl
from jax.experimental.pallas import tpu as pltpu
from jax.experimental.pallas import tpu_sc as plsc

N_NODES = 100000
N_EDGES = 1600000
NUM_GRAPHS = 256
DIM = 50
DIMP = 64
CHUNK = 16
NCHUNK = DIMP // CHUNK
SLOTS = 128 // CHUNK

NC, NS = 2, 16
NW = NC * NS

NPAD = 102400
EBLK = 512
NBLK = 198
EPT = EBLK * NBLK
EPAD = EPT * NS
NAGG = 100352
ETILE = N_EDGES // 128
GROWS = EPAD // NW
BN = 512


def _prep_body(emb_hbm, node_hbm, ea_hbm, web_hbm, x3_hbm, gate_hbm,
               idx_v, rows_v, ea_v, g_v, w_v, sem):
    core = lax.axis_index("c")
    sub = lax.axis_index("s")
    wid = sub * NC + core
    rpt = NPAD // NW

    def blk(b, _):
        base = pl.multiple_of(wid * rpt + b * 128, 128)
        pltpu.sync_copy(node_hbm.at[pl.ds(base, 128)], idx_v)
        pltpu.async_copy(emb_hbm.at[idx_v], rows_v, sem).wait()
        pltpu.sync_copy(rows_v, x3_hbm.at[pl.ds(base, 128), pl.ds(0, DIMP)])
        return 0
    lax.fori_loop(0, rpt // 128, blk, 0)

    pltpu.sync_copy(web_hbm, w_v)
    wv = w_v[...]
    ert = GROWS // 128
    row0 = wid * ert
    nrow = jnp.maximum(jnp.minimum(ETILE - row0, ert), 0)

    def grow(i, _):
        r = row0 + i
        pltpu.sync_copy(ea_hbm.at[r], ea_v)
        for g in range(8):
            z = (ea_v[0, pl.ds(g * 16, 16)] * wv[0]
                 + ea_v[1, pl.ds(g * 16, 16)] * wv[1]
                 + ea_v[2, pl.ds(g * 16, 16)] * wv[2]
                 + ea_v[3, pl.ds(g * 16, 16)] * wv[3]
                 + wv[4])
            g_v[pl.ds(g * 16, 16)] = 1.0 / (1.0 + jnp.exp(-z))
        gbase = pl.multiple_of((row0 + i) * 128, 128)
        pltpu.sync_copy(g_v, gate_hbm.at[pl.ds(gbase, 128)])
        return 0
    lax.fori_loop(0, nrow, grow, 0)

    @pl.when(wid == NW - 1)
    def _():
        def ztail(g, _):
            g_v[pl.ds(g * 16, 16)] = jnp.zeros((16,), jnp.float32)
            return 0
        lax.fori_loop(0, 8, ztail, 0)

        def zrow(i, _):
            gbase = pl.multiple_of(N_EDGES + i * 128, 128)
            pltpu.sync_copy(g_v, gate_hbm.at[pl.ds(gbase, 128)])
            return 0
        lax.fori_loop(0, (EPAD - N_EDGES) // 128, zrow, 0)


def _prep_call(embp, nodep, ea3, web):
    return pl.kernel(
        _prep_body,
        out_type=(
            jax.ShapeDtypeStruct((NPAD, 128), jnp.float32),
            jax.ShapeDtypeStruct((EPAD,), jnp.float32),
        ),
        mesh=plsc.VectorSubcoreMesh(core_axis_name="c", subcore_axis_name="s"),
        compiler_params=pltpu.CompilerParams(use_tc_tiling_on_sc=False),
        scratch_types=[
            pltpu.VMEM((128,), jnp.int32),
            pltpu.VMEM((128, DIMP), jnp.float32),
            pltpu.VMEM((4, 128), jnp.float32),
            pltpu.VMEM((128,), jnp.float32),
            pltpu.VMEM((16,), jnp.float32),
            pltpu.SemaphoreType.DMA,
        ],
        name="sc_prep",
    )(embp, nodep, ea3, web)


def _scatter_body(h_hbm, src_hbm, dst_hbm, gate_hbm, agg_hbm,
                  agg_s, sidx_v, didx_v, gate_v, rows_v, zero_v,
                  si0, si1, si2, sg0, sg1, sg2, ss0, ss1, ss2):
    core = lax.axis_index("c")
    sub = lax.axis_index("s")
    rw = NAGG // NS
    sem_i = (si0, si1, si2)
    sem_g = (sg0, sg1, sg2)
    sem_s = (ss0, ss1, ss2)
    NSUP = NBLK // 3

    def zinit(i, _):
        zero_v[i] = jnp.zeros((CHUNK,), jnp.float32)
        return 0
    lax.fori_loop(0, 64, zinit, 0)

    def idx_copies(b, k):
        ebase = pl.multiple_of(sub * EPT + b * EBLK, EBLK)
        erow = sub * (EPT // 128) + b * (EBLK // 128)
        return (
            pltpu.make_async_copy(src_hbm.at[pl.ds(erow, EBLK // 128)],
                                  sidx_v.at[k], sem_i[k]),
            pltpu.make_async_copy(dst_hbm.at[pl.ds(erow, EBLK // 128)],
                                  didx_v.at[k], sem_i[k]),
            pltpu.make_async_copy(gate_hbm.at[pl.ds(ebase, EBLK)],
                                  gate_v.at[k], sem_i[k]),
        )

    def gather_copies(k):
        return [pltpu.make_async_copy(h_hbm.at[sidx_v.at[k, r]],
                                      rows_v.at[k, pl.ds(r * 128, 128)],
                                      sem_g[k])
                for r in range(EBLK // 128)]

    def scat_copies(k):
        return [pltpu.make_async_copy(rows_v.at[k, pl.ds(r * 128, 128)],
                                      agg_s.at[didx_v.at[k, r]], sem_s[k])
                for r in range(EBLK // 128)]

    for cpass in range(NCHUNK // NC):
        chunk = core * (NCHUNK // NC) + cpass

        def zloop(i, _):
            pltpu.sync_copy(zero_v, agg_s.at[pl.ds(sub * rw + i * 64, 64)])
            return 0
        lax.fori_loop(0, rw // 64, zloop, 0)
        plsc.subcore_barrier()

        def mk_sidx(k):
            for g in range(EBLK // 16):
                r, o = divmod(g, 8)
                sidx_v[k, r, pl.ds(o * 16, 16)] = (
                    sidx_v[k, r, pl.ds(o * 16, 16)] * SLOTS + chunk)

        def stage_next(b, k):
            for cp in idx_copies(b, k):
                cp.wait()
            mk_sidx(k)
            for cp in gather_copies(k):
                cp.start()

        def wait_scat(k):
            for cp in scat_copies(k):
                cp.wait()

        def scale(k):
            def sc16(i, _):
                gbase = pl.multiple_of(i * 16, 16)
                gv = gate_v[k, pl.ds(gbase, 16)]
                for r in range(16):
                    j = gbase + r
                    rows_v[k, j] = rows_v[k, j] * gv[r]
                return 0
            lax.fori_loop(0, EBLK // 16, sc16, 0)

        for cp in idx_copies(0, 0):
            cp.start()
        for cp in idx_copies(1, 1):
            cp.start()
        stage_next(0, 0)

        def sblock(B, _):
            for k in range(3):
                b = B * 3 + k
                s1, s2 = (k + 1) % 3, (k + 2) % 3

                def adv():
                    stage_next(b + 1, s1)
                if k < 2:
                    adv()
                else:
                    pl.when(B < NSUP - 1)(adv)

                for cp in gather_copies(k):
                    cp.wait()
                scale(k)

                def w_s2():
                    wait_scat(s2)
                if k == 0:
                    pl.when(B >= 1)(w_s2)
                else:
                    w_s2()

                def pre2():
                    for cp in idx_copies(b + 2, s2):
                        cp.start()
                if k == 0:
                    pre2()
                else:
                    pl.when(B < NSUP - 1)(pre2)

                for cp in scat_copies(k):
                    cp.start(add=True)
            return 0
        lax.fori_loop(0, NSUP, sblock, 0)
        wait_scat((NBLK - 1) % 3)
        plsc.subcore_barrier()

        r0 = pl.multiple_of(sub * rw, 128)
        c0 = pl.multiple_of(chunk * CHUNK, CHUNK)
        pltpu.sync_copy(agg_s.at[pl.ds(sub * rw, rw)],
                        agg_hbm.at[pl.ds(r0, rw), pl.ds(c0, CHUNK)])
        plsc.subcore_barrier()


def _scatter_call(h128, srcp, dstp, gatep):
    hrows = h128.reshape(SLOTS * NPAD, CHUNK)
    return pl.kernel(
        _scatter_body,
        out_type=jax.ShapeDtypeStruct((NPAD, 128), jnp.float32),
        mesh=plsc.VectorSubcoreMesh(core_axis_name="c", subcore_axis_name="s"),
        compiler_params=pltpu.CompilerParams(use_tc_tiling_on_sc=False),
        scratch_types=[
            pltpu.VMEM_SHARED((NAGG, CHUNK), jnp.float32),
            pltpu.VMEM((3, EBLK // 128, 128), jnp.int32),
            pltpu.VMEM((3, EBLK // 128, 128), jnp.int32),
            pltpu.VMEM((3, EBLK), jnp.float32),
            pltpu.VMEM((3, EBLK, CHUNK), jnp.float32),
            pltpu.VMEM((64, CHUNK), jnp.float32),
        ] + [pltpu.SemaphoreType.DMA] * 9,
        name="sc_edge_scatter",
    )(hrows, srcp, dstp, gatep)


def _layer_body(h_ref, a_ref, ws_ref, wn_ref, b_ref, o_ref):
    h = h_ref[...][:, :DIMP]
    a = a_ref[...][:, :DIMP]
    denom = a[:, DIM:DIM + 1] + 1e-6
    an = a / denom
    z = (jnp.dot(h, ws_ref[...], preferred_element_type=jnp.float32)
         + jnp.dot(an, wn_ref[...], preferred_element_type=jnp.float32)
         + b_ref[...])
    z = jnp.maximum(z, 0.0)
    lanes = lax.broadcasted_iota(jnp.int32, (BN, DIMP), 1)
    z = jnp.where(lanes == DIM, 1.0, z)
    o_ref[...] = jnp.concatenate(
        [z, jnp.zeros((BN, 128 - DIMP), jnp.float32)], axis=1)


def _layer_call(h128, agg128, Wsp, Wnp, bp):
    return pl.pallas_call(
        _layer_body,
        grid=(NPAD // BN,),
        in_specs=[
            pl.BlockSpec((BN, 128), lambda i: (i, 0)),
            pl.BlockSpec((BN, 128), lambda i: (i, 0)),
            pl.BlockSpec((DIMP, DIMP), lambda i: (0, 0)),
            pl.BlockSpec((DIMP, DIMP), lambda i: (0, 0)),
            pl.BlockSpec((1, DIMP), lambda i: (0, 0)),
        ],
        out_specs=pl.BlockSpec((BN, 128), lambda i: (i, 0)),
        out_shape=jax.ShapeDtypeStruct((NPAD, 128), jnp.float32),
    )(h128, agg128, Wsp, Wnp, bp)


def _pool_body(h_ref, b_ref, o_ref):
    i = pl.program_id(0)
    h = h_ref[...][:, :DIMP]
    bid = b_ref[0]
    ohT = (lax.broadcasted_iota(jnp.int32, (NUM_GRAPHS, BN), 0)
           == bid).astype(jnp.float32)
    part = jnp.dot(ohT, h, preferred_element_type=jnp.float32)

    @pl.when(i == 0)
    def _():
        o_ref[...] = part

    @pl.when(i > 0)
    def _():
        o_ref[...] += part


def _pool_call(h128, batch3):
    return pl.pallas_call(
        _pool_body,
        grid=(NPAD // BN,),
        in_specs=[
            pl.BlockSpec((BN, 128), lambda i: (i, 0)),
            pl.BlockSpec((1, 1, BN), lambda i: (i, 0, 0)),
        ],
        out_specs=pl.BlockSpec((NUM_GRAPHS, DIMP), lambda i: (0, 0)),
        out_shape=jax.ShapeDtypeStruct((NUM_GRAPHS, DIMP), jnp.float32),
    )(h128, batch3)


def _logits_body(p_ref, wc_ref, bc_ref, o_ref):
    ps = p_ref[...]
    counts = ps[:, DIM:DIM + 1] + 1e-6
    pm = ps / counts
    o_ref[...] = (jnp.dot(pm, wc_ref[...], preferred_element_type=jnp.float32)
                  + bc_ref[...])


def _logits_call(pooled, Wcp, bcp):
    ncls = Wcp.shape[1]
    return pl.pallas_call(
        _logits_body,
        grid=(1,),
        in_specs=[
            pl.BlockSpec((NUM_GRAPHS, DIMP), lambda i: (0, 0)),
            pl.BlockSpec((DIMP, ncls), lambda i: (0, 0)),
            pl.BlockSpec((1, ncls), lambda i: (0, 0)),
        ],
        out_specs=pl.BlockSpec((NUM_GRAPHS, ncls), lambda i: (0, 0)),
        out_shape=jax.ShapeDtypeStruct((NUM_GRAPHS, ncls), jnp.float32),
    )(pooled, Wcp, bcp)


def _pad64(w):
    return jnp.pad(w, ((0, DIMP - DIM), (0, DIMP - DIM)))


def kernel(node, adj, batch, edge_attr, embedding, We, be,
           W_self1, W_neigh1, b1, W_self2, W_neigh2, b2, W_cls, b_cls):
    V = embedding.shape[0]
    src = adj[0]
    dst = adj[1]

    i32 = jnp.int32
    nodep = jnp.concatenate([node, jnp.zeros((NPAD - N_NODES,), i32)])
    srcp = jnp.concatenate(
        [src, jnp.zeros((EPAD - N_EDGES,), i32)]).reshape(EPAD // 128, 128)
    dstp = jnp.concatenate(
        [dst, jnp.zeros((EPAD - N_EDGES,), i32)]).reshape(EPAD // 128, 128)
    batchp = jnp.concatenate(
        [batch, jnp.full((NPAD - N_NODES,), NUM_GRAPHS, i32)])
    batch3 = batchp.reshape(NPAD // BN, 1, BN)

    embp = jnp.concatenate(
        [embedding, jnp.ones((V, 1), jnp.float32),
         jnp.zeros((V, DIMP - DIM - 1), jnp.float32)], axis=1)
    ea3 = edge_attr.reshape(ETILE, 128, 4).transpose(0, 2, 1)
    web = jnp.concatenate(
        [We.reshape(-1), be, jnp.zeros((16 - 5,), jnp.float32)])

    Wsp1, Wnp1 = _pad64(W_self1), _pad64(W_neigh1)
    Wsp2, Wnp2 = _pad64(W_self2), _pad64(W_neigh2)
    bp1 = jnp.pad(b1, (0, DIMP - DIM)).reshape(1, DIMP)
    bp2 = jnp.pad(b2, (0, DIMP - DIM)).reshape(1, DIMP)
    Wcp = jnp.pad(W_cls, ((0, DIMP - DIM), (0, 0)))
    bcp = b_cls.reshape(1, -1)

    h128, gatep = _prep_call(embp, nodep, ea3, web)
    for Wsp, Wnp, bp in ((Wsp1, Wnp1, bp1), (Wsp2, Wnp2, bp2)):
        agg128 = _scatter_call(h128, srcp, dstp, gatep)
        h128 = _layer_call(h128, agg128, Wsp, Wnp, bp)

    pooled = _pool_call(h128, batch3)
    return _logits_call(pooled, Wcp, bcp)


# speedup vs baseline: 10.0444x; 1.3741x over previous
import jax
import jax.numpy as jnp
from jax import lax
from jax.experimental import pallas as pl
from jax.experimental.pallas import tpu as pltpu
from jax.experimental.pallas import tpu_sc as plsc

N_NODES = 100000
N_EDGES = 1600000
NUM_GRAPHS = 256
DIM = 50
DIMP = 64
CHUNK = 16
NCHUNK = DIMP // CHUNK
SLOTS = 128 // CHUNK

NC, NS = 2, 16
NW = NC * NS

NPAD = 102400
EBLK = 512
NBLK = 198
EPT = EBLK * NBLK
EPAD = EPT * NS
NAGG = 100352
ETILE = N_EDGES // 128
GROWS = EPAD // NW
BN = 512


def _prep_body(emb_hbm, node_hbm, ea_hbm, web_hbm, x3_hbm, gate_hbm,
               idx_v, rows_v, ea_v, g_v, w_v, sem):
    core = lax.axis_index("c")
    sub = lax.axis_index("s")
    wid = sub * NC + core
    rpt = NPAD // NW

    def blk(b, _):
        base = pl.multiple_of(wid * rpt + b * 128, 128)
        pltpu.sync_copy(node_hbm.at[pl.ds(base, 128)], idx_v)
        pltpu.async_copy(emb_hbm.at[idx_v], rows_v, sem).wait()
        pltpu.sync_copy(rows_v, x3_hbm.at[pl.ds(base, 128), pl.ds(0, DIMP)])
        return 0
    lax.fori_loop(0, rpt // 128, blk, 0)

    pltpu.sync_copy(web_hbm, w_v)
    wv = w_v[...]
    ert = GROWS // 128
    row0 = wid * ert
    nrow = jnp.maximum(jnp.minimum(ETILE - row0, ert), 0)

    def grow(i, _):
        r = row0 + i
        pltpu.sync_copy(ea_hbm.at[r], ea_v)
        for g in range(8):
            z = (ea_v[0, pl.ds(g * 16, 16)] * wv[0]
                 + ea_v[1, pl.ds(g * 16, 16)] * wv[1]
                 + ea_v[2, pl.ds(g * 16, 16)] * wv[2]
                 + ea_v[3, pl.ds(g * 16, 16)] * wv[3]
                 + wv[4])
            g_v[pl.ds(g * 16, 16)] = 1.0 / (1.0 + jnp.exp(-z))
        gbase = pl.multiple_of((row0 + i) * 128, 128)
        pltpu.sync_copy(g_v, gate_hbm.at[pl.ds(gbase, 128)])
        return 0
    lax.fori_loop(0, nrow, grow, 0)

    @pl.when(wid == NW - 1)
    def _():
        def ztail(g, _):
            g_v[pl.ds(g * 16, 16)] = jnp.zeros((16,), jnp.float32)
            return 0
        lax.fori_loop(0, 8, ztail, 0)

        def zrow(i, _):
            gbase = pl.multiple_of(N_EDGES + i * 128, 128)
            pltpu.sync_copy(g_v, gate_hbm.at[pl.ds(gbase, 128)])
            return 0
        lax.fori_loop(0, (EPAD - N_EDGES) // 128, zrow, 0)


def _prep_call(embp, nodep, ea3, web):
    return pl.kernel(
        _prep_body,
        out_type=(
            jax.ShapeDtypeStruct((NPAD, 128), jnp.float32),
            jax.ShapeDtypeStruct((EPAD,), jnp.float32),
        ),
        mesh=plsc.VectorSubcoreMesh(core_axis_name="c", subcore_axis_name="s"),
        compiler_params=pltpu.CompilerParams(use_tc_tiling_on_sc=False),
        scratch_types=[
            pltpu.VMEM((128,), jnp.int32),
            pltpu.VMEM((128, DIMP), jnp.float32),
            pltpu.VMEM((4, 128), jnp.float32),
            pltpu.VMEM((128,), jnp.float32),
            pltpu.VMEM((16,), jnp.float32),
            pltpu.SemaphoreType.DMA,
        ],
        name="sc_prep",
    )(embp, nodep, ea3, web)


def _scatter_body(h_hbm, adj_hbm, gate_hbm, agg_hbm,
                  agg_s, sidx_v, didx_v, gate_v, rows_v, zero_v,
                  si0, si1, si2, sg0, sg1, sg2, ss0, ss1, ss2):
    core = lax.axis_index("c")
    sub = lax.axis_index("s")
    rw = NAGG // NS
    sem_i = (si0, si1, si2)
    sem_g = (sg0, sg1, sg2)
    sem_s = (ss0, ss1, ss2)
    NSUP = NBLK // 3

    def zinit(i, _):
        zero_v[i] = jnp.zeros((CHUNK,), jnp.float32)
        return 0
    lax.fori_loop(0, 64, zinit, 0)

    def idx_copies(b, k):
        ebase = pl.multiple_of(sub * EPT + b * EBLK, EBLK)
        erow = sub * (EPT // 128) + b * (EBLK // 128)
        crow = jnp.minimum(erow, ETILE - EBLK // 128)
        return (
            pltpu.make_async_copy(adj_hbm.at[0, pl.ds(crow, EBLK // 128)],
                                  sidx_v.at[k], sem_i[k]),
            pltpu.make_async_copy(adj_hbm.at[1, pl.ds(crow, EBLK // 128)],
                                  didx_v.at[k], sem_i[k]),
            pltpu.make_async_copy(gate_hbm.at[pl.ds(ebase, EBLK)],
                                  gate_v.at[k], sem_i[k]),
        )

    def gather_copies(k):
        return [pltpu.make_async_copy(h_hbm.at[sidx_v.at[k, r]],
                                      rows_v.at[k, pl.ds(r * 128, 128)],
                                      sem_g[k])
                for r in range(EBLK // 128)]

    def scat_copies(k):
        return [pltpu.make_async_copy(rows_v.at[k, pl.ds(r * 128, 128)],
                                      agg_s.at[didx_v.at[k, r]], sem_s[k])
                for r in range(EBLK // 128)]

    for cpass in range(NCHUNK // NC):
        chunk = core * (NCHUNK // NC) + cpass

        def zloop(i, _):
            pltpu.sync_copy(zero_v, agg_s.at[pl.ds(sub * rw + i * 64, 64)])
            return 0
        lax.fori_loop(0, rw // 64, zloop, 0)
        plsc.subcore_barrier()

        def mk_sidx(k):
            for g in range(EBLK // 16):
                r, o = divmod(g, 8)
                sidx_v[k, r, pl.ds(o * 16, 16)] = (
                    sidx_v[k, r, pl.ds(o * 16, 16)] * SLOTS + chunk)

        def stage_next(b, k):
            for cp in idx_copies(b, k):
                cp.wait()
            mk_sidx(k)
            for cp in gather_copies(k):
                cp.start()

        def wait_scat(k):
            for cp in scat_copies(k):
                cp.wait()

        def scale(k):
            @plsc.parallel_loop(0, EBLK // 16, 1, unroll=2)
            def sc16(i):
                gbase = pl.multiple_of(i * 16, 16)
                gv = gate_v[k, pl.ds(gbase, 16)]
                for r in range(16):
                    j = gbase + r
                    rows_v[k, j] = rows_v[k, j] * gv[r]

        for cp in idx_copies(0, 0):
            cp.start()
        for cp in idx_copies(1, 1):
            cp.start()
        stage_next(0, 0)

        def sblock(B, _):
            for k in range(3):
                b = B * 3 + k
                s1, s2 = (k + 1) % 3, (k + 2) % 3

                def adv():
                    stage_next(b + 1, s1)
                if k < 2:
                    adv()
                else:
                    pl.when(B < NSUP - 1)(adv)

                for cp in gather_copies(k):
                    cp.wait()
                scale(k)

                def w_s2():
                    wait_scat(s2)
                if k == 0:
                    pl.when(B >= 1)(w_s2)
                else:
                    w_s2()

                def pre2():
                    for cp in idx_copies(b + 2, s2):
                        cp.start()
                if k == 0:
                    pre2()
                else:
                    pl.when(B < NSUP - 1)(pre2)

                for cp in scat_copies(k):
                    cp.start(add=True)
            return 0
        lax.fori_loop(0, NSUP, sblock, 0)
        wait_scat((NBLK - 1) % 3)
        plsc.subcore_barrier()

        r0 = pl.multiple_of(sub * rw, 128)
        c0 = pl.multiple_of(chunk * CHUNK, CHUNK)
        pltpu.sync_copy(agg_s.at[pl.ds(sub * rw, rw)],
                        agg_hbm.at[pl.ds(r0, rw), pl.ds(c0, CHUNK)])
        plsc.subcore_barrier()


def _scatter_call(h128, adj3, gatep):
    hrows = h128.reshape(SLOTS * NPAD, CHUNK)
    return pl.kernel(
        _scatter_body,
        out_type=jax.ShapeDtypeStruct((NPAD, 128), jnp.float32),
        mesh=plsc.VectorSubcoreMesh(core_axis_name="c", subcore_axis_name="s"),
        compiler_params=pltpu.CompilerParams(use_tc_tiling_on_sc=False),
        scratch_types=[
            pltpu.VMEM_SHARED((NAGG, CHUNK), jnp.float32),
            pltpu.VMEM((3, EBLK // 128, 128), jnp.int32),
            pltpu.VMEM((3, EBLK // 128, 128), jnp.int32),
            pltpu.VMEM((3, EBLK), jnp.float32),
            pltpu.VMEM((3, EBLK, CHUNK), jnp.float32),
            pltpu.VMEM((64, CHUNK), jnp.float32),
        ] + [pltpu.SemaphoreType.DMA] * 9,
        name="sc_edge_scatter",
    )(hrows, adj3, gatep)


def _layer_body(h_ref, a_ref, ws_ref, wn_ref, b_ref, o_ref):
    h = h_ref[...][:, :DIMP]
    a = a_ref[...][:, :DIMP]
    denom = a[:, DIM:DIM + 1] + 1e-6
    an = a / denom
    z = (jnp.dot(h, ws_ref[...], preferred_element_type=jnp.float32)
         + jnp.dot(an, wn_ref[...], preferred_element_type=jnp.float32)
         + b_ref[...])
    z = jnp.maximum(z, 0.0)
    lanes = lax.broadcasted_iota(jnp.int32, (BN, DIMP), 1)
    z = jnp.where(lanes == DIM, 1.0, z)
    o_ref[...] = jnp.concatenate(
        [z, jnp.zeros((BN, 128 - DIMP), jnp.float32)], axis=1)


def _layer_call(h128, agg128, Wsp, Wnp, bp):
    return pl.pallas_call(
        _layer_body,
        grid=(NPAD // BN,),
        in_specs=[
            pl.BlockSpec((BN, 128), lambda i: (i, 0)),
            pl.BlockSpec((BN, 128), lambda i: (i, 0)),
            pl.BlockSpec((DIMP, DIMP), lambda i: (0, 0)),
            pl.BlockSpec((DIMP, DIMP), lambda i: (0, 0)),
            pl.BlockSpec((1, DIMP), lambda i: (0, 0)),
        ],
        out_specs=pl.BlockSpec((BN, 128), lambda i: (i, 0)),
        out_shape=jax.ShapeDtypeStruct((NPAD, 128), jnp.float32),
    )(h128, agg128, Wsp, Wnp, bp)


def _pool_body(h_ref, b_ref, o_ref):
    i = pl.program_id(0)
    h = h_ref[...][:, :DIMP]
    bid = b_ref[0]
    ohT = (lax.broadcasted_iota(jnp.int32, (NUM_GRAPHS, BN), 0)
           == bid).astype(jnp.float32)
    part = jnp.dot(ohT, h, preferred_element_type=jnp.float32)

    @pl.when(i == 0)
    def _():
        o_ref[...] = part

    @pl.when(i > 0)
    def _():
        o_ref[...] += part


def _pool_call(h128, batch3):
    return pl.pallas_call(
        _pool_body,
        grid=(NPAD // BN,),
        in_specs=[
            pl.BlockSpec((BN, 128), lambda i: (i, 0)),
            pl.BlockSpec((1, 1, BN), lambda i: (i, 0, 0)),
        ],
        out_specs=pl.BlockSpec((NUM_GRAPHS, DIMP), lambda i: (0, 0)),
        out_shape=jax.ShapeDtypeStruct((NUM_GRAPHS, DIMP), jnp.float32),
    )(h128, batch3)


def _logits_body(p_ref, wc_ref, bc_ref, o_ref):
    ps = p_ref[...]
    counts = ps[:, DIM:DIM + 1] + 1e-6
    pm = ps / counts
    o_ref[...] = (jnp.dot(pm, wc_ref[...], preferred_element_type=jnp.float32)
                  + bc_ref[...])


def _logits_call(pooled, Wcp, bcp):
    ncls = Wcp.shape[1]
    return pl.pallas_call(
        _logits_body,
        grid=(1,),
        in_specs=[
            pl.BlockSpec((NUM_GRAPHS, DIMP), lambda i: (0, 0)),
            pl.BlockSpec((DIMP, ncls), lambda i: (0, 0)),
            pl.BlockSpec((1, ncls), lambda i: (0, 0)),
        ],
        out_specs=pl.BlockSpec((NUM_GRAPHS, ncls), lambda i: (0, 0)),
        out_shape=jax.ShapeDtypeStruct((NUM_GRAPHS, ncls), jnp.float32),
    )(pooled, Wcp, bcp)


def _pad64(w):
    return jnp.pad(w, ((0, DIMP - DIM), (0, DIMP - DIM)))


def kernel(node, adj, batch, edge_attr, embedding, We, be,
           W_self1, W_neigh1, b1, W_self2, W_neigh2, b2, W_cls, b_cls):
    V = embedding.shape[0]
    src = adj[0]
    dst = adj[1]

    i32 = jnp.int32
    nodep = jnp.concatenate([node, jnp.zeros((NPAD - N_NODES,), i32)])
    adj3 = adj.reshape(2, ETILE, 128)
    batchp = jnp.concatenate(
        [batch, jnp.full((NPAD - N_NODES,), NUM_GRAPHS, i32)])
    batch3 = batchp.reshape(NPAD // BN, 1, BN)

    embp = jnp.concatenate(
        [embedding, jnp.ones((V, 1), jnp.float32),
         jnp.zeros((V, DIMP - DIM - 1), jnp.float32)], axis=1)
    ea3 = edge_attr.reshape(ETILE, 128, 4).transpose(0, 2, 1)
    web = jnp.concatenate(
        [We.reshape(-1), be, jnp.zeros((16 - 5,), jnp.float32)])

    Wsp1, Wnp1 = _pad64(W_self1), _pad64(W_neigh1)
    Wsp2, Wnp2 = _pad64(W_self2), _pad64(W_neigh2)
    bp1 = jnp.pad(b1, (0, DIMP - DIM)).reshape(1, DIMP)
    bp2 = jnp.pad(b2, (0, DIMP - DIM)).reshape(1, DIMP)
    Wcp = jnp.pad(W_cls, ((0, DIMP - DIM), (0, 0)))
    bcp = b_cls.reshape(1, -1)

    h128, gatep = _prep_call(embp, nodep, ea3, web)
    for Wsp, Wnp, bp in ((Wsp1, Wnp1, bp1), (Wsp2, Wnp2, bp2)):
        agg128 = _scatter_call(h128, adj3, gatep)
        h128 = _layer_call(h128, agg128, Wsp, Wnp, bp)

    pooled = _pool_call(h128, batch3)
    return _logits_call(pooled, Wcp, bcp)


# speedup vs baseline: 10.1576x; 1.0113x over previous
import jax
import jax.numpy as jnp
from jax import lax
from jax.experimental import pallas as pl
from jax.experimental.pallas import tpu as pltpu
from jax.experimental.pallas import tpu_sc as plsc

N_NODES = 100000
N_EDGES = 1600000
NUM_GRAPHS = 256
DIM = 50
DIMP = 64
CHUNK = 16
NCHUNK = DIMP // CHUNK
SLOTS = 128 // CHUNK

NC, NS = 2, 16
NW = NC * NS

NPAD = 102400
EBLK = 512
NBLK = 198
EPT = EBLK * NBLK
EPAD = EPT * NS
NAGG = 100352
ETILE = N_EDGES // 128
GROWS = EPAD // NW
BN = 512


def _prep_body(emb_hbm, node_hbm, ea_hbm, web_hbm, x3_hbm, gate_hbm,
               idx_v, rows_v, ea_v, g_v, w_v, sem):
    core = lax.axis_index("c")
    sub = lax.axis_index("s")
    wid = sub * NC + core
    rpt = NPAD // NW

    def blk(b, _):
        base = pl.multiple_of(wid * rpt + b * 128, 128)
        pltpu.sync_copy(node_hbm.at[pl.ds(base, 128)], idx_v)
        pltpu.async_copy(emb_hbm.at[idx_v], rows_v, sem).wait()
        pltpu.sync_copy(rows_v, x3_hbm.at[pl.ds(base, 128), pl.ds(0, DIMP)])
        return 0
    lax.fori_loop(0, rpt // 128, blk, 0)

    pltpu.sync_copy(web_hbm, w_v)
    wv = w_v[...]
    ert = GROWS // 128
    row0 = wid * ert
    nrow = jnp.maximum(jnp.minimum(ETILE - row0, ert), 0)

    def grow(i, _):
        r = row0 + i
        pltpu.sync_copy(ea_hbm.at[r], ea_v)
        for g in range(8):
            z = (ea_v[0, pl.ds(g * 16, 16)] * wv[0]
                 + ea_v[1, pl.ds(g * 16, 16)] * wv[1]
                 + ea_v[2, pl.ds(g * 16, 16)] * wv[2]
                 + ea_v[3, pl.ds(g * 16, 16)] * wv[3]
                 + wv[4])
            g_v[pl.ds(g * 16, 16)] = 1.0 / (1.0 + jnp.exp(-z))
        gbase = pl.multiple_of((row0 + i) * 128, 128)
        pltpu.sync_copy(g_v, gate_hbm.at[pl.ds(gbase, 128)])
        return 0
    lax.fori_loop(0, nrow, grow, 0)

    @pl.when(wid == NW - 1)
    def _():
        def ztail(g, _):
            g_v[pl.ds(g * 16, 16)] = jnp.zeros((16,), jnp.float32)
            return 0
        lax.fori_loop(0, 8, ztail, 0)

        def zrow(i, _):
            gbase = pl.multiple_of(N_EDGES + i * 128, 128)
            pltpu.sync_copy(g_v, gate_hbm.at[pl.ds(gbase, 128)])
            return 0
        lax.fori_loop(0, (EPAD - N_EDGES) // 128, zrow, 0)


def _prep_call(embp, nodep, ea3, web):
    return pl.kernel(
        _prep_body,
        out_type=(
            jax.ShapeDtypeStruct((NPAD, 128), jnp.float32),
            jax.ShapeDtypeStruct((EPAD,), jnp.float32),
        ),
        mesh=plsc.VectorSubcoreMesh(core_axis_name="c", subcore_axis_name="s"),
        compiler_params=pltpu.CompilerParams(use_tc_tiling_on_sc=False),
        scratch_types=[
            pltpu.VMEM((128,), jnp.int32),
            pltpu.VMEM((128, DIMP), jnp.float32),
            pltpu.VMEM((4, 128), jnp.float32),
            pltpu.VMEM((128,), jnp.float32),
            pltpu.VMEM((16,), jnp.float32),
            pltpu.SemaphoreType.DMA,
        ],
        name="sc_prep",
    )(embp, nodep, ea3, web)


def _scatter_body(h_hbm, adj_hbm, gate_hbm, agg_hbm,
                  agg_s, sidx_v, didx_v, gate_v, rows_v, zero_v,
                  si0, si1, si2, sg0, sg1, sg2, ss0, ss1, ss2):
    core = lax.axis_index("c")
    sub = lax.axis_index("s")
    rw = NAGG // NS
    sem_i = (si0, si1, si2)
    sem_g = (sg0, sg1, sg2)
    sem_s = (ss0, ss1, ss2)
    NSUP = NBLK // 3

    def zinit(i, _):
        zero_v[i] = jnp.zeros((CHUNK,), jnp.float32)
        return 0
    lax.fori_loop(0, 64, zinit, 0)

    def idx_copies(b, k):
        ebase = pl.multiple_of(sub * EPT + b * EBLK, EBLK)
        erow = sub * (EPT // 128) + b * (EBLK // 128)
        crow = jnp.minimum(erow, ETILE - EBLK // 128)
        return (
            pltpu.make_async_copy(adj_hbm.at[0, pl.ds(crow, EBLK // 128)],
                                  sidx_v.at[k], sem_i[k]),
            pltpu.make_async_copy(adj_hbm.at[1, pl.ds(crow, EBLK // 128)],
                                  didx_v.at[k], sem_i[k]),
            pltpu.make_async_copy(gate_hbm.at[pl.ds(ebase, EBLK)],
                                  gate_v.at[k], sem_i[k]),
        )

    def gather_copies(k):
        return [pltpu.make_async_copy(h_hbm.at[sidx_v.at[k, r]],
                                      rows_v.at[k, pl.ds(r * 128, 128)],
                                      sem_g[k])
                for r in range(EBLK // 128)]

    def scat_copies(k):
        return [pltpu.make_async_copy(rows_v.at[k, pl.ds(r * 128, 128)],
                                      agg_s.at[didx_v.at[k, r]], sem_s[k])
                for r in range(EBLK // 128)]

    for cpass in range(NCHUNK // NC):
        chunk = core * (NCHUNK // NC) + cpass

        def zloop(i, _):
            pltpu.make_async_copy(
                zero_v, agg_s.at[pl.ds(sub * rw + i * 64, 64)], sg0).start()
            return 0
        lax.fori_loop(0, rw // 64, zloop, 0)

        def zdrain(i, _):
            pltpu.make_async_copy(
                zero_v, agg_s.at[pl.ds(sub * rw + i * 64, 64)], sg0).wait()
            return 0
        lax.fori_loop(0, rw // 64, zdrain, 0)
        plsc.subcore_barrier()

        def mk_sidx(k):
            for g in range(EBLK // 16):
                r, o = divmod(g, 8)
                sidx_v[k, r, pl.ds(o * 16, 16)] = (
                    sidx_v[k, r, pl.ds(o * 16, 16)] * SLOTS + chunk)

        def stage_next(b, k):
            for cp in idx_copies(b, k):
                cp.wait()
            mk_sidx(k)
            for cp in gather_copies(k):
                cp.start()

        def wait_scat(k):
            for cp in scat_copies(k):
                cp.wait()

        def scale(k):
            @plsc.parallel_loop(0, EBLK // 16, 1, unroll=2)
            def sc16(i):
                gbase = pl.multiple_of(i * 16, 16)
                gv = gate_v[k, pl.ds(gbase, 16)]
                for r in range(16):
                    j = gbase + r
                    rows_v[k, j] = rows_v[k, j] * gv[r]

        for cp in idx_copies(0, 0):
            cp.start()
        for cp in idx_copies(1, 1):
            cp.start()
        stage_next(0, 0)

        def sblock(B, _):
            for k in range(3):
                b = B * 3 + k
                s1, s2 = (k + 1) % 3, (k + 2) % 3

                def adv():
                    stage_next(b + 1, s1)
                if k < 2:
                    adv()
                else:
                    pl.when(B < NSUP - 1)(adv)

                for cp in gather_copies(k):
                    cp.wait()
                scale(k)

                def w_s2():
                    wait_scat(s2)
                if k == 0:
                    pl.when(B >= 1)(w_s2)
                else:
                    w_s2()

                def pre2():
                    for cp in idx_copies(b + 2, s2):
                        cp.start()
                if k == 0:
                    pre2()
                else:
                    pl.when(B < NSUP - 1)(pre2)

                for cp in scat_copies(k):
                    cp.start(add=True)
            return 0
        lax.fori_loop(0, NSUP, sblock, 0)
        wait_scat((NBLK - 1) % 3)
        plsc.subcore_barrier()

        r0 = pl.multiple_of(sub * rw, 128)
        c0 = pl.multiple_of(chunk * CHUNK, CHUNK)
        pltpu.sync_copy(agg_s.at[pl.ds(sub * rw, rw)],
                        agg_hbm.at[pl.ds(r0, rw), pl.ds(c0, CHUNK)])
        plsc.subcore_barrier()


def _scatter_call(h128, adj3, gatep):
    hrows = h128.reshape(SLOTS * NPAD, CHUNK)
    return pl.kernel(
        _scatter_body,
        out_type=jax.ShapeDtypeStruct((NPAD, 128), jnp.float32),
        mesh=plsc.VectorSubcoreMesh(core_axis_name="c", subcore_axis_name="s"),
        compiler_params=pltpu.CompilerParams(use_tc_tiling_on_sc=False),
        scratch_types=[
            pltpu.VMEM_SHARED((NAGG, CHUNK), jnp.float32),
            pltpu.VMEM((3, EBLK // 128, 128), jnp.int32),
            pltpu.VMEM((3, EBLK // 128, 128), jnp.int32),
            pltpu.VMEM((3, EBLK), jnp.float32),
            pltpu.VMEM((3, EBLK, CHUNK), jnp.float32),
            pltpu.VMEM((64, CHUNK), jnp.float32),
        ] + [pltpu.SemaphoreType.DMA] * 9,
        name="sc_edge_scatter",
    )(hrows, adj3, gatep)


def _layer_body(h_ref, a_ref, ws_ref, wn_ref, b_ref, o_ref):
    h = h_ref[...][:, :DIMP]
    a = a_ref[...][:, :DIMP]
    denom = a[:, DIM:DIM + 1] + 1e-6
    an = a / denom
    z = (jnp.dot(h, ws_ref[...], preferred_element_type=jnp.float32)
         + jnp.dot(an, wn_ref[...], preferred_element_type=jnp.float32)
         + b_ref[...])
    z = jnp.maximum(z, 0.0)
    lanes = lax.broadcasted_iota(jnp.int32, (BN, DIMP), 1)
    z = jnp.where(lanes == DIM, 1.0, z)
    o_ref[...] = jnp.concatenate(
        [z, jnp.zeros((BN, 128 - DIMP), jnp.float32)], axis=1)


def _layer_call(h128, agg128, Wsp, Wnp, bp):
    return pl.pallas_call(
        _layer_body,
        grid=(NPAD // BN,),
        in_specs=[
            pl.BlockSpec((BN, 128), lambda i: (i, 0)),
            pl.BlockSpec((BN, 128), lambda i: (i, 0)),
            pl.BlockSpec((DIMP, DIMP), lambda i: (0, 0)),
            pl.BlockSpec((DIMP, DIMP), lambda i: (0, 0)),
            pl.BlockSpec((1, DIMP), lambda i: (0, 0)),
        ],
        out_specs=pl.BlockSpec((BN, 128), lambda i: (i, 0)),
        out_shape=jax.ShapeDtypeStruct((NPAD, 128), jnp.float32),
    )(h128, agg128, Wsp, Wnp, bp)


def _pool_body(h_ref, b_ref, o_ref):
    i = pl.program_id(0)
    h = h_ref[...][:, :DIMP]
    bid = b_ref[0]
    ohT = (lax.broadcasted_iota(jnp.int32, (NUM_GRAPHS, BN), 0)
           == bid).astype(jnp.float32)
    part = jnp.dot(ohT, h, preferred_element_type=jnp.float32)

    @pl.when(i == 0)
    def _():
        o_ref[...] = part

    @pl.when(i > 0)
    def _():
        o_ref[...] += part


def _pool_call(h128, batch3):
    return pl.pallas_call(
        _pool_body,
        grid=(NPAD // BN,),
        in_specs=[
            pl.BlockSpec((BN, 128), lambda i: (i, 0)),
            pl.BlockSpec((1, 1, BN), lambda i: (i, 0, 0)),
        ],
        out_specs=pl.BlockSpec((NUM_GRAPHS, DIMP), lambda i: (0, 0)),
        out_shape=jax.ShapeDtypeStruct((NUM_GRAPHS, DIMP), jnp.float32),
    )(h128, batch3)


def _logits_body(p_ref, wc_ref, bc_ref, o_ref):
    ps = p_ref[...]
    counts = ps[:, DIM:DIM + 1] + 1e-6
    pm = ps / counts
    o_ref[...] = (jnp.dot(pm, wc_ref[...], preferred_element_type=jnp.float32)
                  + bc_ref[...])


def _logits_call(pooled, Wcp, bcp):
    ncls = Wcp.shape[1]
    return pl.pallas_call(
        _logits_body,
        grid=(1,),
        in_specs=[
            pl.BlockSpec((NUM_GRAPHS, DIMP), lambda i: (0, 0)),
            pl.BlockSpec((DIMP, ncls), lambda i: (0, 0)),
            pl.BlockSpec((1, ncls), lambda i: (0, 0)),
        ],
        out_specs=pl.BlockSpec((NUM_GRAPHS, ncls), lambda i: (0, 0)),
        out_shape=jax.ShapeDtypeStruct((NUM_GRAPHS, ncls), jnp.float32),
    )(pooled, Wcp, bcp)


def _pad64(w):
    return jnp.pad(w, ((0, DIMP - DIM), (0, DIMP - DIM)))


def kernel(node, adj, batch, edge_attr, embedding, We, be,
           W_self1, W_neigh1, b1, W_self2, W_neigh2, b2, W_cls, b_cls):
    V = embedding.shape[0]
    src = adj[0]
    dst = adj[1]

    i32 = jnp.int32
    nodep = jnp.concatenate([node, jnp.zeros((NPAD - N_NODES,), i32)])
    adj3 = adj.reshape(2, ETILE, 128)
    batchp = jnp.concatenate(
        [batch, jnp.full((NPAD - N_NODES,), NUM_GRAPHS, i32)])
    batch3 = batchp.reshape(NPAD // BN, 1, BN)

    embp = jnp.concatenate(
        [embedding, jnp.ones((V, 1), jnp.float32),
         jnp.zeros((V, DIMP - DIM - 1), jnp.float32)], axis=1)
    ea3 = edge_attr.reshape(ETILE, 128, 4).transpose(0, 2, 1)
    web = jnp.concatenate(
        [We.reshape(-1), be, jnp.zeros((16 - 5,), jnp.float32)])

    Wsp1, Wnp1 = _pad64(W_self1), _pad64(W_neigh1)
    Wsp2, Wnp2 = _pad64(W_self2), _pad64(W_neigh2)
    bp1 = jnp.pad(b1, (0, DIMP - DIM)).reshape(1, DIMP)
    bp2 = jnp.pad(b2, (0, DIMP - DIM)).reshape(1, DIMP)
    Wcp = jnp.pad(W_cls, ((0, DIMP - DIM), (0, 0)))
    bcp = b_cls.reshape(1, -1)

    h128, gatep = _prep_call(embp, nodep, ea3, web)
    for Wsp, Wnp, bp in ((Wsp1, Wnp1, bp1), (Wsp2, Wnp2, bp2)):
        agg128 = _scatter_call(h128, adj3, gatep)
        h128 = _layer_call(h128, agg128, Wsp, Wnp, bp)

    pooled = _pool_call(h128, batch3)
    return _logits_call(pooled, Wcp, bcp)


# speedup vs baseline: 10.7954x; 1.0628x over previous
import jax
import jax.numpy as jnp
from jax import lax
from jax.experimental import pallas as pl
from jax.experimental.pallas import tpu as pltpu
from jax.experimental.pallas import tpu_sc as plsc

N_NODES = 100000
N_EDGES = 1600000
NUM_GRAPHS = 256
DIM = 50
DIMP = 64
CHUNK = 16
NCHUNK = DIMP // CHUNK
SLOTS = 128 // CHUNK

NC, NS = 2, 16
NW = NC * NS

NPAD = 102400
EBLK = 512
NBLK = 198
EPT = EBLK * NBLK
EPAD = EPT * NS
NAGG = 100352
ETILE = N_EDGES // 128
GROWS = EPAD // NW
BN = 512


def _prep_body(emb_hbm, node_hbm, ea_hbm, web_hbm, x3_hbm, gate_hbm,
               idx_v, rows_v, ea_v, g_v, w_v, sem):
    core = lax.axis_index("c")
    sub = lax.axis_index("s")
    wid = sub * NC + core
    rpt = NPAD // NW

    def blk(b, _):
        base = pl.multiple_of(wid * rpt + b * 128, 128)
        pltpu.sync_copy(node_hbm.at[pl.ds(base, 128)], idx_v)
        for g in range(8):
            idx_v[pl.ds(g * 16, 16)] = idx_v[pl.ds(g * 16, 16)] * 2
        pltpu.async_copy(emb_hbm.at[idx_v], rows_v, sem).wait()
        pltpu.sync_copy(rows_v, x3_hbm.at[pl.ds(base, 128), pl.ds(0, DIMP)])
        return 0
    lax.fori_loop(0, rpt // 128, blk, 0)

    pltpu.sync_copy(web_hbm, w_v)
    wv = w_v[...]
    ert = GROWS // 128
    row0 = wid * ert
    nrow = jnp.maximum(jnp.minimum(ETILE - row0, ert), 0)

    def grow(i, _):
        r = row0 + i
        pltpu.sync_copy(ea_hbm.at[r], ea_v)
        for g in range(8):
            z = (ea_v[0, pl.ds(g * 16, 16)] * wv[0]
                 + ea_v[1, pl.ds(g * 16, 16)] * wv[1]
                 + ea_v[2, pl.ds(g * 16, 16)] * wv[2]
                 + ea_v[3, pl.ds(g * 16, 16)] * wv[3]
                 + wv[4])
            g_v[pl.ds(g * 16, 16)] = 1.0 / (1.0 + jnp.exp(-z))
        gbase = pl.multiple_of((row0 + i) * 128, 128)
        pltpu.sync_copy(g_v, gate_hbm.at[pl.ds(gbase, 128)])
        return 0
    lax.fori_loop(0, nrow, grow, 0)

    @pl.when(wid == NW - 1)
    def _():
        def ztail(g, _):
            g_v[pl.ds(g * 16, 16)] = jnp.zeros((16,), jnp.float32)
            return 0
        lax.fori_loop(0, 8, ztail, 0)

        def zrow(i, _):
            gbase = pl.multiple_of(N_EDGES + i * 128, 128)
            pltpu.sync_copy(g_v, gate_hbm.at[pl.ds(gbase, 128)])
            return 0
        lax.fori_loop(0, (EPAD - N_EDGES) // 128, zrow, 0)


def _prep_call(embp, nodep, ea3, web):
    return pl.kernel(
        _prep_body,
        out_type=(
            jax.ShapeDtypeStruct((NPAD, 128), jnp.float32),
            jax.ShapeDtypeStruct((EPAD,), jnp.float32),
        ),
        mesh=plsc.VectorSubcoreMesh(core_axis_name="c", subcore_axis_name="s"),
        compiler_params=pltpu.CompilerParams(use_tc_tiling_on_sc=False),
        scratch_types=[
            pltpu.VMEM((128,), jnp.int32),
            pltpu.VMEM((128, DIMP), jnp.float32),
            pltpu.VMEM((4, 128), jnp.float32),
            pltpu.VMEM((128,), jnp.float32),
            pltpu.VMEM((16,), jnp.float32),
            pltpu.SemaphoreType.DMA,
        ],
        name="sc_prep",
    )(embp, nodep, ea3, web)


def _scatter_body(h_hbm, adj_hbm, gate_hbm, agg_hbm,
                  agg_s, sidx_v, didx_v, gate_v, rows_v, zero_v,
                  si0, si1, si2, sg0, sg1, sg2, ss0, ss1, ss2):
    core = lax.axis_index("c")
    sub = lax.axis_index("s")
    rw = NAGG // NS
    sem_i = (si0, si1, si2)
    sem_g = (sg0, sg1, sg2)
    sem_s = (ss0, ss1, ss2)
    NSUP = NBLK // 3

    def zinit(i, _):
        zero_v[i] = jnp.zeros((CHUNK,), jnp.float32)
        return 0
    lax.fori_loop(0, 64, zinit, 0)

    def idx_copies(b, k):
        ebase = pl.multiple_of(sub * EPT + b * EBLK, EBLK)
        erow = sub * (EPT // 128) + b * (EBLK // 128)
        crow = jnp.minimum(erow, ETILE - EBLK // 128)
        return (
            pltpu.make_async_copy(adj_hbm.at[pl.ds(crow, EBLK // 128), 0],
                                  sidx_v.at[k], sem_i[k]),
            pltpu.make_async_copy(adj_hbm.at[pl.ds(crow, EBLK // 128), 1],
                                  didx_v.at[k], sem_i[k]),
            pltpu.make_async_copy(gate_hbm.at[pl.ds(ebase, EBLK)],
                                  gate_v.at[k], sem_i[k]),
        )

    def gather_copies(k):
        return [pltpu.make_async_copy(h_hbm.at[sidx_v.at[k, r]],
                                      rows_v.at[k, pl.ds(r * 128, 128)],
                                      sem_g[k])
                for r in range(EBLK // 128)]

    def scat_copies(k):
        return [pltpu.make_async_copy(rows_v.at[k, pl.ds(r * 128, 128)],
                                      agg_s.at[didx_v.at[k, r]], sem_s[k])
                for r in range(EBLK // 128)]

    for cpass in range(NCHUNK // NC):
        chunk = core * (NCHUNK // NC) + cpass

        def zloop(i, _):
            pltpu.make_async_copy(
                zero_v, agg_s.at[pl.ds(sub * rw + i * 64, 64)], sg0).start()
            return 0
        lax.fori_loop(0, rw // 64, zloop, 0)

        def zdrain(i, _):
            pltpu.make_async_copy(
                zero_v, agg_s.at[pl.ds(sub * rw + i * 64, 64)], sg0).wait()
            return 0
        lax.fori_loop(0, rw // 64, zdrain, 0)
        plsc.subcore_barrier()

        def mk_sidx(k):
            for g in range(EBLK // 16):
                r, o = divmod(g, 8)
                sidx_v[k, r, pl.ds(o * 16, 16)] = (
                    sidx_v[k, r, pl.ds(o * 16, 16)] * SLOTS + chunk)

        def stage_next(b, k):
            for cp in idx_copies(b, k):
                cp.wait()
            mk_sidx(k)
            for cp in gather_copies(k):
                cp.start()

        def wait_scat(k):
            for cp in scat_copies(k):
                cp.wait()

        def scale(k):
            @plsc.parallel_loop(0, EBLK // 16, 1, unroll=2)
            def sc16(i):
                gbase = pl.multiple_of(i * 16, 16)
                gv = gate_v[k, pl.ds(gbase, 16)]
                for r in range(16):
                    j = gbase + r
                    rows_v[k, j] = rows_v[k, j] * gv[r]

        for cp in idx_copies(0, 0):
            cp.start()
        for cp in idx_copies(1, 1):
            cp.start()
        stage_next(0, 0)

        def sblock(B, _):
            for k in range(3):
                b = B * 3 + k
                s1, s2 = (k + 1) % 3, (k + 2) % 3

                def adv():
                    stage_next(b + 1, s1)
                if k < 2:
                    adv()
                else:
                    pl.when(B < NSUP - 1)(adv)

                for cp in gather_copies(k):
                    cp.wait()
                scale(k)

                def w_s2():
                    wait_scat(s2)
                if k == 0:
                    pl.when(B >= 1)(w_s2)
                else:
                    w_s2()

                def pre2():
                    for cp in idx_copies(b + 2, s2):
                        cp.start()
                if k == 0:
                    pre2()
                else:
                    pl.when(B < NSUP - 1)(pre2)

                for cp in scat_copies(k):
                    cp.start(add=True)
            return 0
        lax.fori_loop(0, NSUP, sblock, 0)
        wait_scat((NBLK - 1) % 3)
        plsc.subcore_barrier()

        r0 = pl.multiple_of(sub * rw, 128)
        c0 = pl.multiple_of(chunk * CHUNK, CHUNK)
        pltpu.sync_copy(agg_s.at[pl.ds(sub * rw, rw)],
                        agg_hbm.at[pl.ds(r0, rw), pl.ds(c0, CHUNK)])
        plsc.subcore_barrier()


def _scatter_call(h128, adj3, gatep):
    hrows = h128.reshape(SLOTS * NPAD, CHUNK)
    return pl.kernel(
        _scatter_body,
        out_type=jax.ShapeDtypeStruct((NPAD, 128), jnp.float32),
        mesh=plsc.VectorSubcoreMesh(core_axis_name="c", subcore_axis_name="s"),
        compiler_params=pltpu.CompilerParams(use_tc_tiling_on_sc=False),
        scratch_types=[
            pltpu.VMEM_SHARED((NAGG, CHUNK), jnp.float32),
            pltpu.VMEM((3, EBLK // 128, 128), jnp.int32),
            pltpu.VMEM((3, EBLK // 128, 128), jnp.int32),
            pltpu.VMEM((3, EBLK), jnp.float32),
            pltpu.VMEM((3, EBLK, CHUNK), jnp.float32),
            pltpu.VMEM((64, CHUNK), jnp.float32),
        ] + [pltpu.SemaphoreType.DMA] * 9,
        name="sc_edge_scatter",
    )(hrows, adj3, gatep)


def _embtab_body(e_ref, o_ref):
    nb = e_ref.shape[0]
    o_ref[...] = jnp.concatenate(
        [e_ref[...], jnp.ones((nb, 1), jnp.float32),
         jnp.zeros((nb, 128 - DIM - 1), jnp.float32)], axis=1)


def _embtab_call(embedding):
    V = embedding.shape[0]
    NB2 = 1000
    return pl.pallas_call(
        _embtab_body,
        grid=(V // NB2,),
        in_specs=[pl.BlockSpec((NB2, DIM), lambda i: (i, 0))],
        out_specs=pl.BlockSpec((NB2, 128), lambda i: (i, 0)),
        out_shape=jax.ShapeDtypeStruct((V, 128), jnp.float32),
    )(embedding)


def _layer_body(h_ref, a_ref, ws_ref, wn_ref, b_ref, o_ref):
    h = h_ref[...][:, :DIMP]
    a = a_ref[...][:, :DIMP]
    denom = a[:, DIM:DIM + 1] + 1e-6
    an = a / denom
    z = (jnp.dot(h, ws_ref[...], preferred_element_type=jnp.float32)
         + jnp.dot(an, wn_ref[...], preferred_element_type=jnp.float32)
         + b_ref[...])
    z = jnp.maximum(z, 0.0)
    lanes = lax.broadcasted_iota(jnp.int32, (BN, DIMP), 1)
    z = jnp.where(lanes == DIM, 1.0, z)
    o_ref[...] = jnp.concatenate(
        [z, jnp.zeros((BN, 128 - DIMP), jnp.float32)], axis=1)


def _layer_call(h128, agg128, Wsp, Wnp, bp):
    return pl.pallas_call(
        _layer_body,
        grid=(NPAD // BN,),
        in_specs=[
            pl.BlockSpec((BN, 128), lambda i: (i, 0)),
            pl.BlockSpec((BN, 128), lambda i: (i, 0)),
            pl.BlockSpec((DIMP, DIMP), lambda i: (0, 0)),
            pl.BlockSpec((DIMP, DIMP), lambda i: (0, 0)),
            pl.BlockSpec((1, DIMP), lambda i: (0, 0)),
        ],
        out_specs=pl.BlockSpec((BN, 128), lambda i: (i, 0)),
        out_shape=jax.ShapeDtypeStruct((NPAD, 128), jnp.float32),
    )(h128, agg128, Wsp, Wnp, bp)


def _layer2_pool_body(h_ref, a_ref, ws_ref, wn_ref, b_ref, bid_ref, o_ref):
    i = pl.program_id(0)
    h = h_ref[...][:, :DIMP]
    a = a_ref[...][:, :DIMP]
    denom = a[:, DIM:DIM + 1] + 1e-6
    an = a / denom
    z = (jnp.dot(h, ws_ref[...], preferred_element_type=jnp.float32)
         + jnp.dot(an, wn_ref[...], preferred_element_type=jnp.float32)
         + b_ref[...])
    z = jnp.maximum(z, 0.0)
    lanes = lax.broadcasted_iota(jnp.int32, (BN, DIMP), 1)
    z = jnp.where(lanes == DIM, 1.0, z)
    bid = bid_ref[0]
    ohT = (lax.broadcasted_iota(jnp.int32, (NUM_GRAPHS, BN), 0)
           == bid).astype(jnp.float32)
    part = jnp.dot(ohT, z, preferred_element_type=jnp.float32)

    @pl.when(i == 0)
    def _():
        o_ref[...] = part

    @pl.when(i > 0)
    def _():
        o_ref[...] += part


def _layer2_pool_call(h128, agg128, Wsp, Wnp, bp, batch3):
    return pl.pallas_call(
        _layer2_pool_body,
        grid=(NPAD // BN,),
        in_specs=[
            pl.BlockSpec((BN, 128), lambda i: (i, 0)),
            pl.BlockSpec((BN, 128), lambda i: (i, 0)),
            pl.BlockSpec((DIMP, DIMP), lambda i: (0, 0)),
            pl.BlockSpec((DIMP, DIMP), lambda i: (0, 0)),
            pl.BlockSpec((1, DIMP), lambda i: (0, 0)),
            pl.BlockSpec((1, 1, BN), lambda i: (i, 0, 0)),
        ],
        out_specs=pl.BlockSpec((NUM_GRAPHS, DIMP), lambda i: (0, 0)),
        out_shape=jax.ShapeDtypeStruct((NUM_GRAPHS, DIMP), jnp.float32),
    )(h128, agg128, Wsp, Wnp, bp, batch3)


def _pool_body(h_ref, b_ref, o_ref):
    i = pl.program_id(0)
    h = h_ref[...][:, :DIMP]
    bid = b_ref[0]
    ohT = (lax.broadcasted_iota(jnp.int32, (NUM_GRAPHS, BN), 0)
           == bid).astype(jnp.float32)
    part = jnp.dot(ohT, h, preferred_element_type=jnp.float32)

    @pl.when(i == 0)
    def _():
        o_ref[...] = part

    @pl.when(i > 0)
    def _():
        o_ref[...] += part


def _pool_call(h128, batch3):
    return pl.pallas_call(
        _pool_body,
        grid=(NPAD // BN,),
        in_specs=[
            pl.BlockSpec((BN, 128), lambda i: (i, 0)),
            pl.BlockSpec((1, 1, BN), lambda i: (i, 0, 0)),
        ],
        out_specs=pl.BlockSpec((NUM_GRAPHS, DIMP), lambda i: (0, 0)),
        out_shape=jax.ShapeDtypeStruct((NUM_GRAPHS, DIMP), jnp.float32),
    )(h128, batch3)


def _logits_body(p_ref, wc_ref, bc_ref, o_ref):
    ps = p_ref[...]
    counts = ps[:, DIM:DIM + 1] + 1e-6
    pm = ps / counts
    o_ref[...] = (jnp.dot(pm, wc_ref[...], preferred_element_type=jnp.float32)
                  + bc_ref[...])


def _logits_call(pooled, Wcp, bcp):
    ncls = Wcp.shape[1]
    return pl.pallas_call(
        _logits_body,
        grid=(1,),
        in_specs=[
            pl.BlockSpec((NUM_GRAPHS, DIMP), lambda i: (0, 0)),
            pl.BlockSpec((DIMP, ncls), lambda i: (0, 0)),
            pl.BlockSpec((1, ncls), lambda i: (0, 0)),
        ],
        out_specs=pl.BlockSpec((NUM_GRAPHS, ncls), lambda i: (0, 0)),
        out_shape=jax.ShapeDtypeStruct((NUM_GRAPHS, ncls), jnp.float32),
    )(pooled, Wcp, bcp)


def _pad64(w):
    return jnp.pad(w, ((0, DIMP - DIM), (0, DIMP - DIM)))


def kernel(node, adj, batch, edge_attr, embedding, We, be,
           W_self1, W_neigh1, b1, W_self2, W_neigh2, b2, W_cls, b_cls):
    V = embedding.shape[0]
    src = adj[0]
    dst = adj[1]

    i32 = jnp.int32
    nodep = jnp.concatenate([node, jnp.zeros((NPAD - N_NODES,), i32)])
    adj3 = adj.reshape(2, ETILE, 128).transpose(1, 0, 2)
    batchp = jnp.concatenate(
        [batch, jnp.full((NPAD - N_NODES,), NUM_GRAPHS, i32)])
    batch3 = batchp.reshape(NPAD // BN, 1, BN)

    embp = _embtab_call(embedding).reshape(2 * V, DIMP)
    ea3 = edge_attr.reshape(ETILE, 128, 4).transpose(0, 2, 1)
    web = jnp.concatenate(
        [We.reshape(-1), be, jnp.zeros((16 - 5,), jnp.float32)])

    Wsp1, Wnp1 = _pad64(W_self1), _pad64(W_neigh1)
    Wsp2, Wnp2 = _pad64(W_self2), _pad64(W_neigh2)
    bp1 = jnp.pad(b1, (0, DIMP - DIM)).reshape(1, DIMP)
    bp2 = jnp.pad(b2, (0, DIMP - DIM)).reshape(1, DIMP)
    Wcp = jnp.pad(W_cls, ((0, DIMP - DIM), (0, 0)))
    bcp = b_cls.reshape(1, -1)

    h128, gatep = _prep_call(embp, nodep, ea3, web)
    agg128 = _scatter_call(h128, adj3, gatep)
    h128 = _layer_call(h128, agg128, Wsp1, Wnp1, bp1)
    agg128 = _scatter_call(h128, adj3, gatep)
    pooled = _layer2_pool_call(h128, agg128, Wsp2, Wnp2, bp2, batch3)
    return _logits_call(pooled, Wcp, bcp)
